# Initial kernel scaffold; baseline (speedup 1.0000x reference)
import jax
import jax.numpy as jnp
from jax.experimental import pallas as pl


def kernel(input_ids, tok_emb, pos_emb, anchor_state, anchor_val, in_norm_w, in_norm_b, sp_w, sp_b, U, V, norm_w, norm_b):
    raise NotImplementedError("write your pallas kernel here")


# speedup vs baseline: 1.3337x; 1.3337x over previous
import functools

import jax
import jax.numpy as jnp
from jax import lax
from jax.experimental import pallas as pl
from jax.experimental.pallas import tpu as pltpu
from jax.experimental.pallas import tpu_sc as plsc

DIM = 768
RANK = 16
WINDOW = 64
BLK = 128
PBLK = 512
EPS = 1e-5


def _ln(x, w, b):
    mu = jnp.mean(x, axis=-1, keepdims=True)
    var = jnp.mean((x - mu) ** 2, axis=-1, keepdims=True)
    return (x - mu) * lax.rsqrt(var + EPS) * w + b


def _sc_gather(table, ids_flat):
    info = plsc.get_sparse_core_info()
    nw = info.num_cores * info.num_subcores
    n = ids_flat.shape[0]
    per_w = n // nw
    mesh = plsc.VectorSubcoreMesh(core_axis_name="c", subcore_axis_name="s")

    @functools.partial(
        pl.kernel,
        mesh=mesh,
        out_type=jax.ShapeDtypeStruct((n, DIM), jnp.float32),
        scratch_types=[
            pltpu.VMEM((per_w,), jnp.int32),
            pltpu.VMEM((per_w, DIM), jnp.float32),
            pltpu.SemaphoreType.DMA,
        ],
    )
    def k(table_hbm, idx_hbm, out_hbm, idx_v, rows_v, sem):
        wid = lax.axis_index("s") * info.num_cores + lax.axis_index("c")
        base = wid * per_w
        pltpu.sync_copy(idx_hbm.at[pl.ds(base, per_w)], idx_v)
        pltpu.async_copy(table_hbm.at[idx_v], rows_v, sem).wait()
        pltpu.sync_copy(rows_v, out_hbm.at[pl.ds(base, per_w)])

    return k(table, ids_flat)


def _prep_body(emb_ref, pos_ref, w_ref, b_ref, out_ref):
    x = emb_ref[...] + pos_ref[...]
    out_ref[...] = _ln(x, w_ref[...], b_ref[...])


def _prep(emb, pos, w, b):
    n = emb.shape[0]
    s = pos.shape[0]
    grid = (n // PBLK,)
    return pl.pallas_call(
        _prep_body,
        grid=grid,
        in_specs=[
            pl.BlockSpec((PBLK, DIM), lambda i: (i, 0)),
            pl.BlockSpec((PBLK, DIM), lambda i: (i % (s // PBLK), 0)),
            pl.BlockSpec((1, DIM), lambda i: (0, 0)),
            pl.BlockSpec((1, DIM), lambda i: (0, 0)),
        ],
        out_specs=pl.BlockSpec((PBLK, DIM), lambda i: (i, 0)),
        out_shape=jax.ShapeDtypeStruct((n, DIM), jnp.float32),
    )(emb, pos, w, b)


def _layer_body(vp_ref, vc_ref, vn_ref, anc_ref, u_ref, v_ref, nw_ref, nb_ref,
                vout_ref, aout_ref, m_s, d_s, acc_ref):
    r = pl.program_id(1)
    nr = pl.num_programs(1)
    s_tokens = nr * BLK

    xc = vc_ref[0]
    vcat = jnp.concatenate([vp_ref[0], xc, vn_ref[0]], axis=0)
    a_row = anc_ref[0]

    u = u_ref[...]
    v = v_ref[...]
    q = jnp.dot(xc, u, preferred_element_type=jnp.float32)
    kcat = jnp.dot(vcat, v, preferred_element_type=jnp.float32)
    k0 = jnp.dot(a_row, v, preferred_element_type=jnp.float32)

    scores = lax.dot_general(q, kcat, (((1,), (1,)), ((), ())),
                             preferred_element_type=jnp.float32) * 0.25
    s0 = lax.dot_general(q, k0, (((1,), (1,)), ((), ())),
                         preferred_element_type=jnp.float32) * 0.25

    ii = r * BLK + lax.broadcasted_iota(jnp.int32, (BLK, 3 * BLK), 0)
    jj = (r - 1) * BLK + lax.broadcasted_iota(jnp.int32, (BLK, 3 * BLK), 1)
    valid = (jnp.abs(ii - jj) <= WINDOW) & (jj >= 0) & (jj < s_tokens)

    asc = jnp.abs(scores)
    m_row = jnp.maximum(
        jnp.max(jnp.where(valid, asc, -jnp.inf), axis=1, keepdims=True),
        jnp.abs(s0))
    e = jnp.where(valid, jnp.exp(asc - m_row), 0.0)
    e0 = jnp.exp(jnp.abs(s0) - m_row)
    denom = jnp.sum(e, axis=1, keepdims=True) + e0
    wgt = jnp.sign(scores) * (e / denom)
    w0 = jnp.sign(s0) * (e0 / denom)

    delta = jnp.dot(wgt, vcat, preferred_element_type=jnp.float32) + w0 * a_row
    nw = nw_ref[...]
    nb = nb_ref[...]
    vout_ref[0] = _ln(xc + delta, nw, nb)

    @pl.when(r == 0)
    def _():
        m_s[0, 0] = -jnp.inf
        d_s[0, 0] = 0.0
        acc_ref[...] = jnp.zeros_like(acc_ref)

    q0 = jnp.dot(a_row, u, preferred_element_type=jnp.float32)
    kc = kcat[BLK:2 * BLK]
    s0r = lax.dot_general(q0, kc, (((1,), (1,)), ((), ())),
                          preferred_element_type=jnp.float32) * 0.25
    a0 = jnp.abs(s0r)
    m_old = m_s[0, 0]
    m_new = jnp.maximum(m_old, jnp.max(a0))
    scale = jnp.exp(m_old - m_new)
    ew = jnp.exp(a0 - m_new)
    d_s[0, 0] = d_s[0, 0] * scale + jnp.sum(ew)
    acc_ref[...] = acc_ref[...] * scale + jnp.dot(
        jnp.sign(s0r) * ew, xc, preferred_element_type=jnp.float32)
    m_s[0, 0] = m_new

    @pl.when(r == nr - 1)
    def _():
        s00 = lax.dot_general(q0, k0, (((1,), (1,)), ((), ())),
                              preferred_element_type=jnp.float32)[0, 0] * 0.25
        a00 = jnp.abs(s00)
        m_old2 = m_s[0, 0]
        m_f = jnp.maximum(m_old2, a00)
        sc2 = jnp.exp(m_old2 - m_f)
        e00 = jnp.exp(a00 - m_f)
        d_f = d_s[0, 0] * sc2 + e00
        acc_f = acc_ref[...] * sc2 + jnp.sign(s00) * e00 * a_row
        aout_ref[0] = _ln(a_row + acc_f / d_f, nw, nb)


def _layer(vmain, anchor, u, v, nw, nb):
    b, s, _ = vmain.shape
    r = s // BLK
    grid = (b, r)
    return pl.pallas_call(
        _layer_body,
        grid=grid,
        in_specs=[
            pl.BlockSpec((1, BLK, DIM), lambda bi, ri: (bi, jnp.maximum(ri - 1, 0), 0)),
            pl.BlockSpec((1, BLK, DIM), lambda bi, ri: (bi, ri, 0)),
            pl.BlockSpec((1, BLK, DIM),
                         lambda bi, ri, _r=r: (bi, jnp.minimum(ri + 1, _r - 1), 0)),
            pl.BlockSpec((1, 1, DIM), lambda bi, ri: (bi, 0, 0)),
            pl.BlockSpec((DIM, RANK), lambda bi, ri: (0, 0)),
            pl.BlockSpec((DIM, RANK), lambda bi, ri: (0, 0)),
            pl.BlockSpec((1, DIM), lambda bi, ri: (0, 0)),
            pl.BlockSpec((1, DIM), lambda bi, ri: (0, 0)),
        ],
        out_specs=[
            pl.BlockSpec((1, BLK, DIM), lambda bi, ri: (bi, ri, 0)),
            pl.BlockSpec((1, 1, DIM), lambda bi, ri: (bi, 0, 0)),
        ],
        out_shape=[
            jax.ShapeDtypeStruct((b, s, DIM), jnp.float32),
            jax.ShapeDtypeStruct((b, 1, DIM), jnp.float32),
        ],
        scratch_shapes=[
            pltpu.SMEM((1, 1), jnp.float32),
            pltpu.SMEM((1, 1), jnp.float32),
            pltpu.VMEM((1, DIM), jnp.float32),
        ],
        compiler_params=pltpu.CompilerParams(
            dimension_semantics=("arbitrary", "arbitrary")),
    )(vmain, vmain, vmain, anchor, u, v, nw, nb)


def kernel(input_ids, tok_emb, pos_emb, anchor_state, anchor_val, in_norm_w,
           in_norm_b, sp_w, sp_b, U, V, norm_w, norm_b):
    del anchor_state, sp_w, sp_b
    bsz, seq = input_ids.shape
    ids = input_ids.reshape(-1).astype(jnp.int32)
    emb = _sc_gather(tok_emb, ids)
    vmain = _prep(emb, pos_emb[:seq],
                  in_norm_w.reshape(1, DIM), in_norm_b.reshape(1, DIM))
    vmain = vmain.reshape(bsz, seq, DIM)
    anchor = jnp.broadcast_to(anchor_val.reshape(1, 1, DIM), (bsz, 1, DIM))
    anchor = jnp.asarray(anchor)
    for l in range(U.shape[0]):
        vmain, anchor = _layer(vmain, anchor, U[l, 0], V[l, 0],
                               norm_w[l].reshape(1, DIM),
                               norm_b[l].reshape(1, DIM))
    return jnp.concatenate([anchor, vmain], axis=1)


# speedup vs baseline: 1.6805x; 1.2600x over previous
import functools

import jax
import jax.numpy as jnp
from jax import lax
from jax.experimental import pallas as pl
from jax.experimental.pallas import tpu as pltpu
from jax.experimental.pallas import tpu_sc as plsc

DIM = 768
RANK = 16
WINDOW = 64
BLK = 128
PBLK = 512
EPS = 1e-5


def _ln(x, w, b):
    mu = jnp.mean(x, axis=-1, keepdims=True)
    var = jnp.mean((x - mu) ** 2, axis=-1, keepdims=True)
    return (x - mu) * lax.rsqrt(var + EPS) * w + b


def _sc_gather(table, ids_flat):
    info = plsc.get_sparse_core_info()
    nw = info.num_cores * info.num_subcores
    n = ids_flat.shape[0]
    per_w = n // nw
    mesh = plsc.VectorSubcoreMesh(core_axis_name="c", subcore_axis_name="s")

    @functools.partial(
        pl.kernel,
        mesh=mesh,
        out_type=jax.ShapeDtypeStruct((n, DIM), jnp.float32),
        scratch_types=[
            pltpu.VMEM((per_w,), jnp.int32),
            pltpu.VMEM((per_w, DIM), jnp.float32),
            pltpu.SemaphoreType.DMA,
        ],
    )
    def k(table_hbm, idx_hbm, out_hbm, idx_v, rows_v, sem):
        wid = lax.axis_index("s") * info.num_cores + lax.axis_index("c")
        base = wid * per_w
        pltpu.sync_copy(idx_hbm.at[pl.ds(base, per_w)], idx_v)
        pltpu.async_copy(table_hbm.at[idx_v], rows_v, sem).wait()
        pltpu.sync_copy(rows_v, out_hbm.at[pl.ds(base, per_w)])

    return k(table, ids_flat)


def _prep_body(emb_ref, pos_ref, w_ref, b_ref, out_ref):
    x = emb_ref[...] + pos_ref[...]
    out_ref[...] = _ln(x, w_ref[...], b_ref[...])


def _prep(emb, pos, w, b):
    n = emb.shape[0]
    s = pos.shape[0]
    grid = (n // PBLK,)
    return pl.pallas_call(
        _prep_body,
        grid=grid,
        in_specs=[
            pl.BlockSpec((PBLK, DIM), lambda i: (i, 0)),
            pl.BlockSpec((PBLK, DIM), lambda i: (i % (s // PBLK), 0)),
            pl.BlockSpec((1, DIM), lambda i: (0, 0)),
            pl.BlockSpec((1, DIM), lambda i: (0, 0)),
        ],
        out_specs=pl.BlockSpec((PBLK, DIM), lambda i: (i, 0)),
        out_shape=jax.ShapeDtypeStruct((n, DIM), jnp.float32),
    )(emb, pos, w, b)


def _layer_body(*refs, final):
    if final:
        (vp_ref, vc_ref, vn_ref, anc_ref, u_ref, v_ref, nw_ref, nb_ref,
         vout_ref, m_s, d_s, acc_ref, prev_ref) = refs
        aout_ref = None
    else:
        (vp_ref, vc_ref, vn_ref, anc_ref, u_ref, v_ref, nw_ref, nb_ref,
         vout_ref, aout_ref, m_s, d_s, acc_ref) = refs
    r = pl.program_id(1)
    nr = pl.num_programs(1)
    s_tokens = nr * BLK

    xc = vc_ref[0]
    vcat = jnp.concatenate([vp_ref[0], xc, vn_ref[0]], axis=0)
    a_row = anc_ref[0]

    u = u_ref[...]
    v = v_ref[...]
    q = jnp.dot(xc, u, preferred_element_type=jnp.float32)
    kcat = jnp.dot(vcat, v, preferred_element_type=jnp.float32)
    k0 = jnp.dot(a_row, v, preferred_element_type=jnp.float32)

    scores = lax.dot_general(q, kcat, (((1,), (1,)), ((), ())),
                             preferred_element_type=jnp.float32) * 0.25
    s0 = lax.dot_general(q, k0, (((1,), (1,)), ((), ())),
                         preferred_element_type=jnp.float32) * 0.25

    ii = r * BLK + lax.broadcasted_iota(jnp.int32, (BLK, 3 * BLK), 0)
    jj = (r - 1) * BLK + lax.broadcasted_iota(jnp.int32, (BLK, 3 * BLK), 1)
    valid = (jnp.abs(ii - jj) <= WINDOW) & (jj >= 0) & (jj < s_tokens)

    asc = jnp.abs(scores)
    m_row = jnp.maximum(
        jnp.max(jnp.where(valid, asc, -jnp.inf), axis=1, keepdims=True),
        jnp.abs(s0))
    e = jnp.where(valid, jnp.exp(asc - m_row), 0.0)
    e0 = jnp.exp(jnp.abs(s0) - m_row)
    denom = jnp.sum(e, axis=1, keepdims=True) + e0
    wgt = jnp.sign(scores) * (e / denom)
    w0 = jnp.sign(s0) * (e0 / denom)

    delta = jnp.dot(wgt, vcat, preferred_element_type=jnp.float32) + w0 * a_row
    nw = nw_ref[...]
    nb = nb_ref[...]
    y = _ln(xc + delta, nw, nb)
    if final:
        shifted = jnp.concatenate([prev_ref[...], y[:BLK - 1]], axis=0)
        prev_ref[...] = y[BLK - 1:]
        vout_ref[0, pl.ds(r * BLK, BLK), :] = shifted
        @pl.when(r == pl.num_programs(1) - 1)
        def _():
            vout_ref[0, BLK * pl.num_programs(1):, :] = y[BLK - 1:]
    else:
        vout_ref[0] = y

    @pl.when(r == 0)
    def _():
        m_s[0, 0] = -jnp.inf
        d_s[0, 0] = 0.0
        acc_ref[...] = jnp.zeros_like(acc_ref)

    q0 = jnp.dot(a_row, u, preferred_element_type=jnp.float32)
    kc = kcat[BLK:2 * BLK]
    s0r = lax.dot_general(q0, kc, (((1,), (1,)), ((), ())),
                          preferred_element_type=jnp.float32) * 0.25
    a0 = jnp.abs(s0r)
    m_old = m_s[0, 0]
    m_new = jnp.maximum(m_old, jnp.max(a0))
    scale = jnp.exp(m_old - m_new)
    ew = jnp.exp(a0 - m_new)
    d_s[0, 0] = d_s[0, 0] * scale + jnp.sum(ew)
    acc_ref[...] = acc_ref[...] * scale + jnp.dot(
        jnp.sign(s0r) * ew, xc, preferred_element_type=jnp.float32)
    m_s[0, 0] = m_new

    @pl.when(r == nr - 1)
    def _():
        s00 = lax.dot_general(q0, k0, (((1,), (1,)), ((), ())),
                              preferred_element_type=jnp.float32)[0, 0] * 0.25
        a00 = jnp.abs(s00)
        m_old2 = m_s[0, 0]
        m_f = jnp.maximum(m_old2, a00)
        sc2 = jnp.exp(m_old2 - m_f)
        e00 = jnp.exp(a00 - m_f)
        d_f = d_s[0, 0] * sc2 + e00
        acc_f = acc_ref[...] * sc2 + jnp.sign(s00) * e00 * a_row
        a_out = _ln(a_row + acc_f / d_f, nw, nb)
        if final:
            vout_ref[0, 0:1, :] = a_out
        else:
            aout_ref[0] = a_out


def _layer(vmain, anchor, u, v, nw, nb, final=False):
    b, s, _ = vmain.shape
    r = s // BLK
    grid = (b, r)
    if final:
        out_specs = pl.BlockSpec((1, s + 1, DIM), lambda bi, ri: (bi, 0, 0))
        out_shape = jax.ShapeDtypeStruct((b, s + 1, DIM), jnp.float32)
    else:
        out_specs = [
            pl.BlockSpec((1, BLK, DIM), lambda bi, ri: (bi, ri, 0)),
            pl.BlockSpec((1, 1, DIM), lambda bi, ri: (bi, 0, 0)),
        ]
        out_shape = [
            jax.ShapeDtypeStruct((b, s, DIM), jnp.float32),
            jax.ShapeDtypeStruct((b, 1, DIM), jnp.float32),
        ]
    return pl.pallas_call(
        functools.partial(_layer_body, final=final),
        grid=grid,
        in_specs=[
            pl.BlockSpec((1, BLK, DIM), lambda bi, ri: (bi, jnp.maximum(ri - 1, 0), 0)),
            pl.BlockSpec((1, BLK, DIM), lambda bi, ri: (bi, ri, 0)),
            pl.BlockSpec((1, BLK, DIM),
                         lambda bi, ri, _r=r: (bi, jnp.minimum(ri + 1, _r - 1), 0)),
            pl.BlockSpec((1, 1, DIM), lambda bi, ri: (bi, 0, 0)),
            pl.BlockSpec((DIM, RANK), lambda bi, ri: (0, 0)),
            pl.BlockSpec((DIM, RANK), lambda bi, ri: (0, 0)),
            pl.BlockSpec((1, DIM), lambda bi, ri: (0, 0)),
            pl.BlockSpec((1, DIM), lambda bi, ri: (0, 0)),
        ],
        out_specs=out_specs,
        out_shape=out_shape,
        scratch_shapes=(
            [pltpu.SMEM((1, 1), jnp.float32),
             pltpu.SMEM((1, 1), jnp.float32),
             pltpu.VMEM((1, DIM), jnp.float32)]
            + ([pltpu.VMEM((1, DIM), jnp.float32)] if final else [])
        ),
        compiler_params=pltpu.CompilerParams(
            dimension_semantics=("arbitrary", "arbitrary")),
    )(vmain, vmain, vmain, anchor, u, v, nw, nb)


def kernel(input_ids, tok_emb, pos_emb, anchor_state, anchor_val, in_norm_w,
           in_norm_b, sp_w, sp_b, U, V, norm_w, norm_b):
    del anchor_state, sp_w, sp_b
    bsz, seq = input_ids.shape
    ids = input_ids.reshape(-1).astype(jnp.int32)
    emb = _sc_gather(tok_emb, ids)
    vmain = _prep(emb, pos_emb[:seq],
                  in_norm_w.reshape(1, DIM), in_norm_b.reshape(1, DIM))
    vmain = vmain.reshape(bsz, seq, DIM)
    anchor = jnp.broadcast_to(anchor_val.reshape(1, 1, DIM), (bsz, 1, DIM))
    anchor = jnp.asarray(anchor)
    nlayers = U.shape[0]
    for l in range(nlayers - 1):
        vmain, anchor = _layer(vmain, anchor, U[l, 0], V[l, 0],
                               norm_w[l].reshape(1, DIM),
                               norm_b[l].reshape(1, DIM))
    l = nlayers - 1
    return _layer(vmain, anchor, U[l, 0], V[l, 0],
                  norm_w[l].reshape(1, DIM), norm_b[l].reshape(1, DIM),
                  final=True)


# speedup vs baseline: 1.7908x; 1.0656x over previous
import functools

import jax
import jax.numpy as jnp
from jax import lax
from jax.experimental import pallas as pl
from jax.experimental.pallas import tpu as pltpu
from jax.experimental.pallas import tpu_sc as plsc

DIM = 768
RANK = 16
WINDOW = 64
BLK = 128
HALF = 64
PBLK = 512
EPS = 1e-5


def _ln(x, w, b):
    mu = jnp.mean(x, axis=-1, keepdims=True)
    var = jnp.mean((x - mu) ** 2, axis=-1, keepdims=True)
    return (x - mu) * lax.rsqrt(var + EPS) * w + b


def _sc_gather(table, ids_flat):
    info = plsc.get_sparse_core_info()
    nw = info.num_cores * info.num_subcores
    n = ids_flat.shape[0]
    per_w = n // nw
    mesh = plsc.VectorSubcoreMesh(core_axis_name="c", subcore_axis_name="s")

    @functools.partial(
        pl.kernel,
        mesh=mesh,
        out_type=jax.ShapeDtypeStruct((n, DIM), jnp.float32),
        scratch_types=[
            pltpu.VMEM((per_w,), jnp.int32),
            pltpu.VMEM((per_w, DIM), jnp.float32),
            pltpu.SemaphoreType.DMA,
        ],
    )
    def k(table_hbm, idx_hbm, out_hbm, idx_v, rows_v, sem):
        wid = lax.axis_index("s") * info.num_cores + lax.axis_index("c")
        base = wid * per_w
        pltpu.sync_copy(idx_hbm.at[pl.ds(base, per_w)], idx_v)
        pltpu.async_copy(table_hbm.at[idx_v], rows_v, sem).wait()
        pltpu.sync_copy(rows_v, out_hbm.at[pl.ds(base, per_w)])

    return k(table, ids_flat)


def _prep_body(emb_ref, pos_ref, w_ref, b_ref, out_ref):
    x = emb_ref[...] + pos_ref[...]
    out_ref[...] = _ln(x, w_ref[...], b_ref[...])


def _prep(emb, pos, w, b):
    n = emb.shape[0]
    s = pos.shape[0]
    grid = (n // PBLK,)
    return pl.pallas_call(
        _prep_body,
        grid=grid,
        in_specs=[
            pl.BlockSpec((PBLK, DIM), lambda i: (i, 0)),
            pl.BlockSpec((PBLK, DIM), lambda i: (i % (s // PBLK), 0)),
            pl.BlockSpec((1, DIM), lambda i: (0, 0)),
            pl.BlockSpec((1, DIM), lambda i: (0, 0)),
        ],
        out_specs=pl.BlockSpec((PBLK, DIM), lambda i: (i, 0)),
        out_shape=jax.ShapeDtypeStruct((n, DIM), jnp.float32),
    )(emb, pos, w, b)


def _layer_body(*refs, final):
    if final:
        (p0_ref, p1_ref, p2_ref, p3_ref, anc_ref, u_ref, v_ref, nw_ref, nb_ref,
         vout_ref, m_s, d_s, acc_ref, prev_ref) = refs
        aout_ref = None
    else:
        (p0_ref, p1_ref, p2_ref, p3_ref, anc_ref, u_ref, v_ref, nw_ref, nb_ref,
         vout_ref, aout_ref, m_s, d_s, acc_ref) = refs
    r = pl.program_id(1)
    nr = pl.num_programs(1)
    s_tokens = nr * BLK

    vcat = jnp.concatenate(
        [p0_ref[0], p1_ref[0], p2_ref[0], p3_ref[0]], axis=0)
    xc = vcat[HALF:HALF + BLK]
    a_row = anc_ref[0]

    u = u_ref[...]
    v = v_ref[...]
    q = jnp.dot(xc, u, preferred_element_type=jnp.float32)
    kcat = jnp.dot(vcat, v, preferred_element_type=jnp.float32)
    k0 = jnp.dot(a_row, v, preferred_element_type=jnp.float32)

    scores = lax.dot_general(q, kcat, (((1,), (1,)), ((), ())),
                             preferred_element_type=jnp.float32) * 0.25
    s0 = lax.dot_general(q, k0, (((1,), (1,)), ((), ())),
                         preferred_element_type=jnp.float32) * 0.25

    ii = r * BLK + lax.broadcasted_iota(jnp.int32, (BLK, 2 * BLK), 0)
    jj = r * BLK - HALF + lax.broadcasted_iota(jnp.int32, (BLK, 2 * BLK), 1)
    valid = (jnp.abs(ii - jj) <= WINDOW) & (jj >= 0) & (jj < s_tokens)

    asc = jnp.abs(scores)
    m_row = jnp.maximum(
        jnp.max(jnp.where(valid, asc, -jnp.inf), axis=1, keepdims=True),
        jnp.abs(s0))
    e = jnp.where(valid, jnp.exp(asc - m_row), 0.0)
    e0 = jnp.exp(jnp.abs(s0) - m_row)
    denom = jnp.sum(e, axis=1, keepdims=True) + e0
    wgt = jnp.sign(scores) * (e / denom)
    w0 = jnp.sign(s0) * (e0 / denom)

    delta = jnp.dot(wgt, vcat, preferred_element_type=jnp.float32) + w0 * a_row
    nw = nw_ref[...]
    nb = nb_ref[...]
    y = _ln(xc + delta, nw, nb)
    if final:
        shifted = jnp.concatenate([prev_ref[...], y[:BLK - 1]], axis=0)
        prev_ref[...] = y[BLK - 1:]
        vout_ref[0, pl.ds(r * BLK, BLK), :] = shifted
        @pl.when(r == pl.num_programs(1) - 1)
        def _():
            vout_ref[0, BLK * pl.num_programs(1):, :] = y[BLK - 1:]
    else:
        vout_ref[0] = y

    @pl.when(r == 0)
    def _():
        m_s[0, 0] = -jnp.inf
        d_s[0, 0] = 0.0
        acc_ref[...] = jnp.zeros_like(acc_ref)

    q0 = jnp.dot(a_row, u, preferred_element_type=jnp.float32)
    kc = kcat[HALF:HALF + BLK]
    s0r = lax.dot_general(q0, kc, (((1,), (1,)), ((), ())),
                          preferred_element_type=jnp.float32) * 0.25
    a0 = jnp.abs(s0r)
    m_old = m_s[0, 0]
    m_new = jnp.maximum(m_old, jnp.max(a0))
    scale = jnp.exp(m_old - m_new)
    ew = jnp.exp(a0 - m_new)
    d_s[0, 0] = d_s[0, 0] * scale + jnp.sum(ew)
    acc_ref[...] = acc_ref[...] * scale + jnp.dot(
        jnp.sign(s0r) * ew, xc, preferred_element_type=jnp.float32)
    m_s[0, 0] = m_new

    @pl.when(r == nr - 1)
    def _():
        s00 = lax.dot_general(q0, k0, (((1,), (1,)), ((), ())),
                              preferred_element_type=jnp.float32)[0, 0] * 0.25
        a00 = jnp.abs(s00)
        m_old2 = m_s[0, 0]
        m_f = jnp.maximum(m_old2, a00)
        sc2 = jnp.exp(m_old2 - m_f)
        e00 = jnp.exp(a00 - m_f)
        d_f = d_s[0, 0] * sc2 + e00
        acc_f = acc_ref[...] * sc2 + jnp.sign(s00) * e00 * a_row
        a_out = _ln(a_row + acc_f / d_f, nw, nb)
        if final:
            vout_ref[0, 0:1, :] = a_out
        else:
            aout_ref[0] = a_out


def _layer(vmain, anchor, u, v, nw, nb, final=False):
    b, s, _ = vmain.shape
    r = s // BLK
    grid = (b, r)
    if final:
        out_specs = pl.BlockSpec((1, s + 1, DIM), lambda bi, ri: (bi, 0, 0))
        out_shape = jax.ShapeDtypeStruct((b, s + 1, DIM), jnp.float32)
    else:
        out_specs = [
            pl.BlockSpec((1, BLK, DIM), lambda bi, ri: (bi, ri, 0)),
            pl.BlockSpec((1, 1, DIM), lambda bi, ri: (bi, 0, 0)),
        ]
        out_shape = [
            jax.ShapeDtypeStruct((b, s, DIM), jnp.float32),
            jax.ShapeDtypeStruct((b, 1, DIM), jnp.float32),
        ]
    return pl.pallas_call(
        functools.partial(_layer_body, final=final),
        grid=grid,
        in_specs=[
            pl.BlockSpec((1, HALF, DIM),
                         lambda bi, ri, _k=k, _nh=s // HALF:
                         (bi, jnp.clip(2 * ri - 1 + _k, 0, _nh - 1), 0))
            for k in range(4)
        ] + [
            pl.BlockSpec((1, 1, DIM), lambda bi, ri: (bi, 0, 0)),
            pl.BlockSpec((DIM, RANK), lambda bi, ri: (0, 0)),
            pl.BlockSpec((DIM, RANK), lambda bi, ri: (0, 0)),
            pl.BlockSpec((1, DIM), lambda bi, ri: (0, 0)),
            pl.BlockSpec((1, DIM), lambda bi, ri: (0, 0)),
        ],
        out_specs=out_specs,
        out_shape=out_shape,
        scratch_shapes=(
            [pltpu.SMEM((1, 1), jnp.float32),
             pltpu.SMEM((1, 1), jnp.float32),
             pltpu.VMEM((1, DIM), jnp.float32)]
            + ([pltpu.VMEM((1, DIM), jnp.float32)] if final else [])
        ),
        compiler_params=pltpu.CompilerParams(
            dimension_semantics=("arbitrary", "arbitrary")),
    )(vmain, vmain, vmain, vmain, anchor, u, v, nw, nb)


def kernel(input_ids, tok_emb, pos_emb, anchor_state, anchor_val, in_norm_w,
           in_norm_b, sp_w, sp_b, U, V, norm_w, norm_b):
    del anchor_state, sp_w, sp_b
    bsz, seq = input_ids.shape
    ids = input_ids.reshape(-1).astype(jnp.int32)
    emb = _sc_gather(tok_emb, ids)
    vmain = _prep(emb, pos_emb[:seq],
                  in_norm_w.reshape(1, DIM), in_norm_b.reshape(1, DIM))
    vmain = vmain.reshape(bsz, seq, DIM)
    anchor = jnp.broadcast_to(anchor_val.reshape(1, 1, DIM), (bsz, 1, DIM))
    anchor = jnp.asarray(anchor)
    nlayers = U.shape[0]
    for l in range(nlayers - 1):
        vmain, anchor = _layer(vmain, anchor, U[l, 0], V[l, 0],
                               norm_w[l].reshape(1, DIM),
                               norm_b[l].reshape(1, DIM))
    l = nlayers - 1
    return _layer(vmain, anchor, U[l, 0], V[l, 0],
                  norm_w[l].reshape(1, DIM), norm_b[l].reshape(1, DIM),
                  final=True)


# speedup vs baseline: 1.8426x; 1.0290x over previous
import functools

import jax
import jax.numpy as jnp
from jax import lax
from jax.experimental import pallas as pl
from jax.experimental.pallas import tpu as pltpu
from jax.experimental.pallas import tpu_sc as plsc

DIM = 768
RANK = 16
WINDOW = 64
BLK = 128
HALF = 64
PBLK = 512
EPS = 1e-5


def _ln(x, w, b):
    mu = jnp.mean(x, axis=-1, keepdims=True)
    var = jnp.mean((x - mu) ** 2, axis=-1, keepdims=True)
    return (x - mu) * lax.rsqrt(var + EPS) * w + b


def _sc_gather(table, ids_flat):
    info = plsc.get_sparse_core_info()
    nw = info.num_cores * info.num_subcores
    n = ids_flat.shape[0]
    per_w = n // nw
    mesh = plsc.VectorSubcoreMesh(core_axis_name="c", subcore_axis_name="s")

    @functools.partial(
        pl.kernel,
        mesh=mesh,
        out_type=jax.ShapeDtypeStruct((n, DIM), jnp.float32),
        scratch_types=[
            pltpu.VMEM((per_w,), jnp.int32),
            pltpu.VMEM((per_w, DIM), jnp.float32),
            pltpu.SemaphoreType.DMA,
        ],
    )
    def k(table_hbm, idx_hbm, out_hbm, idx_v, rows_v, sem):
        wid = lax.axis_index("s") * info.num_cores + lax.axis_index("c")
        base = wid * per_w
        pltpu.sync_copy(idx_hbm.at[pl.ds(base, per_w)], idx_v)
        pltpu.async_copy(table_hbm.at[idx_v], rows_v, sem).wait()
        pltpu.sync_copy(rows_v, out_hbm.at[pl.ds(base, per_w)])

    return k(table, ids_flat)


def _prep_body(emb_ref, pos_ref, w_ref, b_ref, out_ref):
    x = emb_ref[...] + pos_ref[...]
    out_ref[...] = _ln(x, w_ref[...], b_ref[...])


def _prep(emb, pos, w, b):
    n = emb.shape[0]
    s = pos.shape[0]
    grid = (n // PBLK,)
    return pl.pallas_call(
        _prep_body,
        grid=grid,
        in_specs=[
            pl.BlockSpec((PBLK, DIM), lambda i: (i, 0)),
            pl.BlockSpec((PBLK, DIM), lambda i: (i % (s // PBLK), 0)),
            pl.BlockSpec((1, DIM), lambda i: (0, 0)),
            pl.BlockSpec((1, DIM), lambda i: (0, 0)),
        ],
        out_specs=pl.BlockSpec((PBLK, DIM), lambda i: (i, 0)),
        out_shape=jax.ShapeDtypeStruct((n, DIM), jnp.float32),
    )(emb, pos, w, b)


def _layer_body(*refs, final, first):
    refs = list(refs)
    panel_refs = [refs.pop(0) for _ in range(4)]
    if first:
        pos_refs = [refs.pop(0) for _ in range(4)]
        inw_ref = refs.pop(0)
        inb_ref = refs.pop(0)
    if final:
        (anc_ref, u_ref, v_ref, nw_ref, nb_ref,
         vout_ref, m_s, d_s, acc_ref, prev_ref) = refs
        aout_ref = None
    else:
        (anc_ref, u_ref, v_ref, nw_ref, nb_ref,
         vout_ref, aout_ref, m_s, d_s, acc_ref) = refs
    r = pl.program_id(1)
    nr = pl.num_programs(1)
    s_tokens = nr * BLK

    if first:
        panels = [
            _ln(p[0] + q[...], inw_ref[...], inb_ref[...])
            for p, q in zip(panel_refs, pos_refs)
        ]
    else:
        panels = [p[0] for p in panel_refs]
    vcat = jnp.concatenate(panels, axis=0)
    xc = vcat[HALF:HALF + BLK]
    a_row = anc_ref[0]

    u = u_ref[...]
    v = v_ref[...]
    q = jnp.dot(xc, u, preferred_element_type=jnp.float32)
    kcat = jnp.dot(vcat, v, preferred_element_type=jnp.float32)
    k0 = jnp.dot(a_row, v, preferred_element_type=jnp.float32)

    scores = lax.dot_general(q, kcat, (((1,), (1,)), ((), ())),
                             preferred_element_type=jnp.float32) * 0.25
    s0 = lax.dot_general(q, k0, (((1,), (1,)), ((), ())),
                         preferred_element_type=jnp.float32) * 0.25

    ii = r * BLK + lax.broadcasted_iota(jnp.int32, (BLK, 2 * BLK), 0)
    jj = r * BLK - HALF + lax.broadcasted_iota(jnp.int32, (BLK, 2 * BLK), 1)
    valid = (jnp.abs(ii - jj) <= WINDOW) & (jj >= 0) & (jj < s_tokens)

    asc = jnp.abs(scores)
    m_row = jnp.maximum(
        jnp.max(jnp.where(valid, asc, -jnp.inf), axis=1, keepdims=True),
        jnp.abs(s0))
    e = jnp.where(valid, jnp.exp(asc - m_row), 0.0)
    e0 = jnp.exp(jnp.abs(s0) - m_row)
    denom = jnp.sum(e, axis=1, keepdims=True) + e0
    wgt = jnp.sign(scores) * (e / denom)
    w0 = jnp.sign(s0) * (e0 / denom)

    delta = jnp.dot(wgt, vcat, preferred_element_type=jnp.float32) + w0 * a_row
    nw = nw_ref[...]
    nb = nb_ref[...]
    y = _ln(xc + delta, nw, nb)
    if final:
        shifted = jnp.concatenate([prev_ref[...], y[:BLK - 1]], axis=0)
        prev_ref[...] = y[BLK - 1:]
        vout_ref[0, pl.ds(r * BLK, BLK), :] = shifted
        @pl.when(r == pl.num_programs(1) - 1)
        def _():
            vout_ref[0, BLK * pl.num_programs(1):, :] = y[BLK - 1:]
    else:
        vout_ref[0] = y

    @pl.when(r == 0)
    def _():
        m_s[0, 0] = -jnp.inf
        d_s[0, 0] = 0.0
        acc_ref[...] = jnp.zeros_like(acc_ref)

    q0 = jnp.dot(a_row, u, preferred_element_type=jnp.float32)
    kc = kcat[HALF:HALF + BLK]
    s0r = lax.dot_general(q0, kc, (((1,), (1,)), ((), ())),
                          preferred_element_type=jnp.float32) * 0.25
    a0 = jnp.abs(s0r)
    m_old = m_s[0, 0]
    m_new = jnp.maximum(m_old, jnp.max(a0))
    scale = jnp.exp(m_old - m_new)
    ew = jnp.exp(a0 - m_new)
    d_s[0, 0] = d_s[0, 0] * scale + jnp.sum(ew)
    acc_ref[...] = acc_ref[...] * scale + jnp.dot(
        jnp.sign(s0r) * ew, xc, preferred_element_type=jnp.float32)
    m_s[0, 0] = m_new

    @pl.when(r == nr - 1)
    def _():
        s00 = lax.dot_general(q0, k0, (((1,), (1,)), ((), ())),
                              preferred_element_type=jnp.float32)[0, 0] * 0.25
        a00 = jnp.abs(s00)
        m_old2 = m_s[0, 0]
        m_f = jnp.maximum(m_old2, a00)
        sc2 = jnp.exp(m_old2 - m_f)
        e00 = jnp.exp(a00 - m_f)
        d_f = d_s[0, 0] * sc2 + e00
        acc_f = acc_ref[...] * sc2 + jnp.sign(s00) * e00 * a_row
        a_out = _ln(a_row + acc_f / d_f, nw, nb)
        if final:
            vout_ref[0, 0:1, :] = a_out
        else:
            aout_ref[0] = a_out


def _layer(vmain, anchor, u, v, nw, nb, final=False, prep=None):
    b, s, _ = vmain.shape
    r = s // BLK
    grid = (b, r)
    if final:
        out_specs = pl.BlockSpec((1, s + 1, DIM), lambda bi, ri: (bi, 0, 0))
        out_shape = jax.ShapeDtypeStruct((b, s + 1, DIM), jnp.float32)
    else:
        out_specs = [
            pl.BlockSpec((1, BLK, DIM), lambda bi, ri: (bi, ri, 0)),
            pl.BlockSpec((1, 1, DIM), lambda bi, ri: (bi, 0, 0)),
        ]
        out_shape = [
            jax.ShapeDtypeStruct((b, s, DIM), jnp.float32),
            jax.ShapeDtypeStruct((b, 1, DIM), jnp.float32),
        ]
    panel_specs = [
        pl.BlockSpec((1, HALF, DIM),
                     lambda bi, ri, _k=k, _nh=s // HALF:
                     (bi, jnp.clip(2 * ri - 1 + _k, 0, _nh - 1), 0))
        for k in range(4)
    ]
    first = prep is not None
    extra_in = []
    extra_specs = []
    if first:
        pos, inw, inb = prep
        extra_in = [pos, pos, pos, pos, inw, inb]
        extra_specs = [
            pl.BlockSpec((HALF, DIM),
                         lambda bi, ri, _k=k, _nh=s // HALF:
                         (jnp.clip(2 * ri - 1 + _k, 0, _nh - 1), 0))
            for k in range(4)
        ] + [
            pl.BlockSpec((1, DIM), lambda bi, ri: (0, 0)),
            pl.BlockSpec((1, DIM), lambda bi, ri: (0, 0)),
        ]
    return pl.pallas_call(
        functools.partial(_layer_body, final=final, first=first),
        grid=grid,
        in_specs=panel_specs + extra_specs + [
            pl.BlockSpec((1, 1, DIM), lambda bi, ri: (bi, 0, 0)),
            pl.BlockSpec((DIM, RANK), lambda bi, ri: (0, 0)),
            pl.BlockSpec((DIM, RANK), lambda bi, ri: (0, 0)),
            pl.BlockSpec((1, DIM), lambda bi, ri: (0, 0)),
            pl.BlockSpec((1, DIM), lambda bi, ri: (0, 0)),
        ],
        out_specs=out_specs,
        out_shape=out_shape,
        scratch_shapes=(
            [pltpu.SMEM((1, 1), jnp.float32),
             pltpu.SMEM((1, 1), jnp.float32),
             pltpu.VMEM((1, DIM), jnp.float32)]
            + ([pltpu.VMEM((1, DIM), jnp.float32)] if final else [])
        ),
        compiler_params=pltpu.CompilerParams(
            dimension_semantics=("arbitrary", "arbitrary")),
    )(vmain, vmain, vmain, vmain, *extra_in, anchor, u, v, nw, nb)


def kernel(input_ids, tok_emb, pos_emb, anchor_state, anchor_val, in_norm_w,
           in_norm_b, sp_w, sp_b, U, V, norm_w, norm_b):
    del anchor_state, sp_w, sp_b
    bsz, seq = input_ids.shape
    ids = input_ids.reshape(-1).astype(jnp.int32)
    emb = _sc_gather(tok_emb, ids)
    vmain = emb.reshape(bsz, seq, DIM)
    anchor = jnp.broadcast_to(anchor_val.reshape(1, 1, DIM), (bsz, 1, DIM))
    anchor = jnp.asarray(anchor)
    nlayers = U.shape[0]
    prep = (pos_emb[:seq], in_norm_w.reshape(1, DIM), in_norm_b.reshape(1, DIM))
    for l in range(nlayers - 1):
        vmain, anchor = _layer(vmain, anchor, U[l, 0], V[l, 0],
                               norm_w[l].reshape(1, DIM),
                               norm_b[l].reshape(1, DIM),
                               prep=prep if l == 0 else None)
    l = nlayers - 1
    return _layer(vmain, anchor, U[l, 0], V[l, 0],
                  norm_w[l].reshape(1, DIM), norm_b[l].reshape(1, DIM),
                  final=True, prep=prep if nlayers == 1 else None)


# speedup vs baseline: 2.3774x; 1.2902x over previous
import functools

import jax
import jax.numpy as jnp
from jax import lax
from jax.experimental import pallas as pl
from jax.experimental.pallas import tpu as pltpu
from jax.experimental.pallas import tpu_sc as plsc

DIM = 768
RANK = 16
WINDOW = 64
BLK = 256
HALF = 64
WIDTH = BLK + 2 * HALF
NPANEL = WIDTH // HALF
PBLK = 512
EPS = 1e-5


def _ln(x, w, b):
    mu = jnp.mean(x, axis=-1, keepdims=True)
    var = jnp.mean((x - mu) ** 2, axis=-1, keepdims=True)
    return (x - mu) * lax.rsqrt(var + EPS) * w + b


def _sc_gather(table, ids_flat):
    info = plsc.get_sparse_core_info()
    nw = info.num_cores * info.num_subcores
    n = ids_flat.shape[0]
    per_w = n // nw
    mesh = plsc.VectorSubcoreMesh(core_axis_name="c", subcore_axis_name="s")

    @functools.partial(
        pl.kernel,
        mesh=mesh,
        out_type=jax.ShapeDtypeStruct((n, DIM), jnp.float32),
        scratch_types=[
            pltpu.VMEM((per_w,), jnp.int32),
            pltpu.VMEM((per_w, DIM), jnp.float32),
            pltpu.SemaphoreType.DMA,
        ],
    )
    def k(table_hbm, idx_hbm, out_hbm, idx_v, rows_v, sem):
        wid = lax.axis_index("s") * info.num_cores + lax.axis_index("c")
        base = wid * per_w
        pltpu.sync_copy(idx_hbm.at[pl.ds(base, per_w)], idx_v)
        pltpu.async_copy(table_hbm.at[idx_v], rows_v, sem).wait()
        pltpu.sync_copy(rows_v, out_hbm.at[pl.ds(base, per_w)])

    return k(table, ids_flat)


def _prep_body(emb_ref, pos_ref, w_ref, b_ref, out_ref):
    x = emb_ref[...] + pos_ref[...]
    out_ref[...] = _ln(x, w_ref[...], b_ref[...])


def _prep(emb, pos, w, b):
    n = emb.shape[0]
    s = pos.shape[0]
    grid = (n // PBLK,)
    return pl.pallas_call(
        _prep_body,
        grid=grid,
        in_specs=[
            pl.BlockSpec((PBLK, DIM), lambda i: (i, 0)),
            pl.BlockSpec((PBLK, DIM), lambda i: (i % (s // PBLK), 0)),
            pl.BlockSpec((1, DIM), lambda i: (0, 0)),
            pl.BlockSpec((1, DIM), lambda i: (0, 0)),
        ],
        out_specs=pl.BlockSpec((PBLK, DIM), lambda i: (i, 0)),
        out_shape=jax.ShapeDtypeStruct((n, DIM), jnp.float32),
    )(emb, pos, w, b)


def _layer_body(*refs, final, first):
    refs = list(refs)
    panel_refs = [refs.pop(0) for _ in range(NPANEL)]
    if first:
        pos_refs = [refs.pop(0) for _ in range(NPANEL)]
        inw_ref = refs.pop(0)
        inb_ref = refs.pop(0)
    if final:
        (anc_ref, u_ref, v_ref, nw_ref, nb_ref,
         vout_ref, m_s, d_s, acc_ref, prev_ref) = refs
        aout_ref = None
    else:
        (anc_ref, u_ref, v_ref, nw_ref, nb_ref,
         vout_ref, aout_ref, m_s, d_s, acc_ref) = refs
    r = pl.program_id(1)
    nr = pl.num_programs(1)
    s_tokens = nr * BLK

    if first:
        panels = [
            _ln(p[0] + q[...], inw_ref[...], inb_ref[...])
            for p, q in zip(panel_refs, pos_refs)
        ]
    else:
        panels = [p[0] for p in panel_refs]
    vcat = jnp.concatenate(panels, axis=0)
    xc = vcat[HALF:HALF + BLK]
    a_row = anc_ref[0]

    u = u_ref[...]
    v = v_ref[...]
    q = jnp.dot(xc, u, preferred_element_type=jnp.float32)
    kcat = jnp.dot(vcat, v, preferred_element_type=jnp.float32)
    k0 = jnp.dot(a_row, v, preferred_element_type=jnp.float32)

    scores = lax.dot_general(q, kcat, (((1,), (1,)), ((), ())),
                             preferred_element_type=jnp.float32) * 0.25
    s0 = lax.dot_general(q, k0, (((1,), (1,)), ((), ())),
                         preferred_element_type=jnp.float32) * 0.25

    ii = r * BLK + lax.broadcasted_iota(jnp.int32, (BLK, WIDTH), 0)
    jj = r * BLK - HALF + lax.broadcasted_iota(jnp.int32, (BLK, WIDTH), 1)
    valid = (jnp.abs(ii - jj) <= WINDOW) & (jj >= 0) & (jj < s_tokens)

    asc = jnp.abs(scores)
    m_row = jnp.maximum(
        jnp.max(jnp.where(valid, asc, -jnp.inf), axis=1, keepdims=True),
        jnp.abs(s0))
    e = jnp.where(valid, jnp.exp(asc - m_row), 0.0)
    e0 = jnp.exp(jnp.abs(s0) - m_row)
    denom = jnp.sum(e, axis=1, keepdims=True) + e0
    wgt = jnp.sign(scores) * (e / denom)
    w0 = jnp.sign(s0) * (e0 / denom)

    delta = jnp.dot(wgt, vcat, preferred_element_type=jnp.float32) + w0 * a_row
    nw = nw_ref[...]
    nb = nb_ref[...]
    y = _ln(xc + delta, nw, nb)
    if final:
        shifted = jnp.concatenate([prev_ref[...], y[:BLK - 1]], axis=0)
        prev_ref[...] = y[BLK - 1:]
        vout_ref[0, pl.ds(r * BLK, BLK), :] = shifted
        @pl.when(r == pl.num_programs(1) - 1)
        def _():
            vout_ref[0, BLK * pl.num_programs(1):, :] = y[BLK - 1:]
    else:
        vout_ref[0] = y

    @pl.when(r == 0)
    def _():
        m_s[0, 0] = -jnp.inf
        d_s[0, 0] = 0.0
        acc_ref[...] = jnp.zeros_like(acc_ref)

    q0 = jnp.dot(a_row, u, preferred_element_type=jnp.float32)
    kc = kcat[HALF:HALF + BLK]
    s0r = lax.dot_general(q0, kc, (((1,), (1,)), ((), ())),
                          preferred_element_type=jnp.float32) * 0.25
    a0 = jnp.abs(s0r)
    m_old = m_s[0, 0]
    m_new = jnp.maximum(m_old, jnp.max(a0))
    scale = jnp.exp(m_old - m_new)
    ew = jnp.exp(a0 - m_new)
    d_s[0, 0] = d_s[0, 0] * scale + jnp.sum(ew)
    acc_ref[...] = acc_ref[...] * scale + jnp.dot(
        jnp.sign(s0r) * ew, xc, preferred_element_type=jnp.float32)
    m_s[0, 0] = m_new

    @pl.when(r == nr - 1)
    def _():
        s00 = lax.dot_general(q0, k0, (((1,), (1,)), ((), ())),
                              preferred_element_type=jnp.float32)[0, 0] * 0.25
        a00 = jnp.abs(s00)
        m_old2 = m_s[0, 0]
        m_f = jnp.maximum(m_old2, a00)
        sc2 = jnp.exp(m_old2 - m_f)
        e00 = jnp.exp(a00 - m_f)
        d_f = d_s[0, 0] * sc2 + e00
        acc_f = acc_ref[...] * sc2 + jnp.sign(s00) * e00 * a_row
        a_out = _ln(a_row + acc_f / d_f, nw, nb)
        if final:
            vout_ref[0, 0:1, :] = a_out
        else:
            aout_ref[0] = a_out


def _layer(vmain, anchor, u, v, nw, nb, final=False, prep=None):
    b, s, _ = vmain.shape
    r = s // BLK
    grid = (b, r)
    if final:
        out_specs = pl.BlockSpec((1, s + 1, DIM), lambda bi, ri: (bi, 0, 0))
        out_shape = jax.ShapeDtypeStruct((b, s + 1, DIM), jnp.float32)
    else:
        out_specs = [
            pl.BlockSpec((1, BLK, DIM), lambda bi, ri: (bi, ri, 0)),
            pl.BlockSpec((1, 1, DIM), lambda bi, ri: (bi, 0, 0)),
        ]
        out_shape = [
            jax.ShapeDtypeStruct((b, s, DIM), jnp.float32),
            jax.ShapeDtypeStruct((b, 1, DIM), jnp.float32),
        ]
    panel_specs = [
        pl.BlockSpec((1, HALF, DIM),
                     lambda bi, ri, _k=k, _nh=s // HALF:
                     (bi, jnp.clip(ri * (BLK // HALF) - 1 + _k, 0, _nh - 1), 0))
        for k in range(NPANEL)
    ]
    first = prep is not None
    extra_in = []
    extra_specs = []
    if first:
        pos, inw, inb = prep
        extra_in = [pos] * NPANEL + [inw, inb]
        extra_specs = [
            pl.BlockSpec((HALF, DIM),
                         lambda bi, ri, _k=k, _nh=s // HALF:
                         (jnp.clip(ri * (BLK // HALF) - 1 + _k, 0, _nh - 1), 0))
            for k in range(NPANEL)
        ] + [
            pl.BlockSpec((1, DIM), lambda bi, ri: (0, 0)),
            pl.BlockSpec((1, DIM), lambda bi, ri: (0, 0)),
        ]
    return pl.pallas_call(
        functools.partial(_layer_body, final=final, first=first),
        grid=grid,
        in_specs=panel_specs + extra_specs + [
            pl.BlockSpec((1, 1, DIM), lambda bi, ri: (bi, 0, 0)),
            pl.BlockSpec((DIM, RANK), lambda bi, ri: (0, 0)),
            pl.BlockSpec((DIM, RANK), lambda bi, ri: (0, 0)),
            pl.BlockSpec((1, DIM), lambda bi, ri: (0, 0)),
            pl.BlockSpec((1, DIM), lambda bi, ri: (0, 0)),
        ],
        out_specs=out_specs,
        out_shape=out_shape,
        scratch_shapes=(
            [pltpu.SMEM((1, 1), jnp.float32),
             pltpu.SMEM((1, 1), jnp.float32),
             pltpu.VMEM((1, DIM), jnp.float32)]
            + ([pltpu.VMEM((1, DIM), jnp.float32)] if final else [])
        ),
        compiler_params=pltpu.CompilerParams(
            dimension_semantics=("arbitrary", "arbitrary")),
    )(*([vmain] * NPANEL), *extra_in, anchor, u, v, nw, nb)


def kernel(input_ids, tok_emb, pos_emb, anchor_state, anchor_val, in_norm_w,
           in_norm_b, sp_w, sp_b, U, V, norm_w, norm_b):
    del anchor_state, sp_w, sp_b
    bsz, seq = input_ids.shape
    ids = input_ids.reshape(-1).astype(jnp.int32)
    emb = _sc_gather(tok_emb, ids)
    vmain = emb.reshape(bsz, seq, DIM)
    anchor = jnp.broadcast_to(anchor_val.reshape(1, 1, DIM), (bsz, 1, DIM))
    anchor = jnp.asarray(anchor)
    nlayers = U.shape[0]
    prep = (pos_emb[:seq], in_norm_w.reshape(1, DIM), in_norm_b.reshape(1, DIM))
    for l in range(nlayers - 1):
        vmain, anchor = _layer(vmain, anchor, U[l, 0], V[l, 0],
                               norm_w[l].reshape(1, DIM),
                               norm_b[l].reshape(1, DIM),
                               prep=prep if l == 0 else None)
    l = nlayers - 1
    return _layer(vmain, anchor, U[l, 0], V[l, 0],
                  norm_w[l].reshape(1, DIM), norm_b[l].reshape(1, DIM),
                  final=True, prep=prep if nlayers == 1 else None)


# speedup vs baseline: 2.5293x; 1.0639x over previous
import functools

import jax
import jax.numpy as jnp
from jax import lax
from jax.experimental import pallas as pl
from jax.experimental.pallas import tpu as pltpu
from jax.experimental.pallas import tpu_sc as plsc

DIM = 768
RANK = 16
WINDOW = 64
BLK = 512
HALF = 64
WIDTH = BLK + 2 * HALF
NPANEL = WIDTH // HALF
PBLK = 512
EPS = 1e-5


def _ln(x, w, b):
    mu = jnp.mean(x, axis=-1, keepdims=True)
    var = jnp.mean((x - mu) ** 2, axis=-1, keepdims=True)
    return (x - mu) * lax.rsqrt(var + EPS) * w + b


def _sc_gather(table, ids_flat):
    info = plsc.get_sparse_core_info()
    nw = info.num_cores * info.num_subcores
    n = ids_flat.shape[0]
    per_w = n // nw
    mesh = plsc.VectorSubcoreMesh(core_axis_name="c", subcore_axis_name="s")

    @functools.partial(
        pl.kernel,
        mesh=mesh,
        out_type=jax.ShapeDtypeStruct((n, DIM), jnp.float32),
        scratch_types=[
            pltpu.VMEM((per_w,), jnp.int32),
            pltpu.VMEM((per_w, DIM), jnp.float32),
            pltpu.SemaphoreType.DMA,
        ],
    )
    def k(table_hbm, idx_hbm, out_hbm, idx_v, rows_v, sem):
        wid = lax.axis_index("s") * info.num_cores + lax.axis_index("c")
        base = wid * per_w
        pltpu.sync_copy(idx_hbm.at[pl.ds(base, per_w)], idx_v)
        pltpu.async_copy(table_hbm.at[idx_v], rows_v, sem).wait()
        pltpu.sync_copy(rows_v, out_hbm.at[pl.ds(base, per_w)])

    return k(table, ids_flat)


def _prep_body(emb_ref, pos_ref, w_ref, b_ref, out_ref):
    x = emb_ref[...] + pos_ref[...]
    out_ref[...] = _ln(x, w_ref[...], b_ref[...])


def _prep(emb, pos, w, b):
    n = emb.shape[0]
    s = pos.shape[0]
    grid = (n // PBLK,)
    return pl.pallas_call(
        _prep_body,
        grid=grid,
        in_specs=[
            pl.BlockSpec((PBLK, DIM), lambda i: (i, 0)),
            pl.BlockSpec((PBLK, DIM), lambda i: (i % (s // PBLK), 0)),
            pl.BlockSpec((1, DIM), lambda i: (0, 0)),
            pl.BlockSpec((1, DIM), lambda i: (0, 0)),
        ],
        out_specs=pl.BlockSpec((PBLK, DIM), lambda i: (i, 0)),
        out_shape=jax.ShapeDtypeStruct((n, DIM), jnp.float32),
    )(emb, pos, w, b)


def _layer_body(*refs, final, first):
    refs = list(refs)
    panel_refs = [refs.pop(0) for _ in range(NPANEL)]
    if first:
        pos_refs = [refs.pop(0) for _ in range(NPANEL)]
        inw_ref = refs.pop(0)
        inb_ref = refs.pop(0)
    if final:
        (anc_ref, u_ref, v_ref, nw_ref, nb_ref,
         vout_ref, m_s, d_s, acc_ref, prev_ref) = refs
        aout_ref = None
    else:
        (anc_ref, u_ref, v_ref, nw_ref, nb_ref,
         vout_ref, aout_ref, m_s, d_s, acc_ref) = refs
    r = pl.program_id(1)
    nr = pl.num_programs(1)
    s_tokens = nr * BLK

    if first:
        panels = [
            _ln(p[0] + q[...], inw_ref[...], inb_ref[...])
            for p, q in zip(panel_refs, pos_refs)
        ]
    else:
        panels = [p[0] for p in panel_refs]
    vcat = jnp.concatenate(panels, axis=0)
    xc = vcat[HALF:HALF + BLK]
    a_row = anc_ref[0]

    u = u_ref[...]
    v = v_ref[...]
    q = jnp.dot(xc, u, preferred_element_type=jnp.float32)
    kcat = jnp.dot(vcat, v, preferred_element_type=jnp.float32)
    k0 = jnp.dot(a_row, v, preferred_element_type=jnp.float32)

    scores = lax.dot_general(q, kcat, (((1,), (1,)), ((), ())),
                             preferred_element_type=jnp.float32) * 0.25
    s0 = lax.dot_general(q, k0, (((1,), (1,)), ((), ())),
                         preferred_element_type=jnp.float32) * 0.25

    ii = r * BLK + lax.broadcasted_iota(jnp.int32, (BLK, WIDTH), 0)
    jj = r * BLK - HALF + lax.broadcasted_iota(jnp.int32, (BLK, WIDTH), 1)
    valid = (jnp.abs(ii - jj) <= WINDOW) & (jj >= 0) & (jj < s_tokens)

    asc = jnp.abs(scores)
    m_row = jnp.maximum(
        jnp.max(jnp.where(valid, asc, -jnp.inf), axis=1, keepdims=True),
        jnp.abs(s0))
    e = jnp.where(valid, jnp.exp(asc - m_row), 0.0)
    e0 = jnp.exp(jnp.abs(s0) - m_row)
    denom = jnp.sum(e, axis=1, keepdims=True) + e0
    wgt = jnp.sign(scores) * (e / denom)
    w0 = jnp.sign(s0) * (e0 / denom)

    delta = jnp.dot(wgt, vcat, preferred_element_type=jnp.float32) + w0 * a_row
    nw = nw_ref[...]
    nb = nb_ref[...]
    y = _ln(xc + delta, nw, nb)
    if final:
        shifted = jnp.concatenate([prev_ref[...], y[:BLK - 1]], axis=0)
        prev_ref[...] = y[BLK - 1:]
        vout_ref[0, pl.ds(r * BLK, BLK), :] = shifted
        @pl.when(r == pl.num_programs(1) - 1)
        def _():
            vout_ref[0, BLK * pl.num_programs(1):, :] = y[BLK - 1:]
    else:
        vout_ref[0] = y

    @pl.when(r == 0)
    def _():
        m_s[0, 0] = -jnp.inf
        d_s[0, 0] = 0.0
        acc_ref[...] = jnp.zeros_like(acc_ref)

    q0 = jnp.dot(a_row, u, preferred_element_type=jnp.float32)
    kc = kcat[HALF:HALF + BLK]
    s0r = lax.dot_general(q0, kc, (((1,), (1,)), ((), ())),
                          preferred_element_type=jnp.float32) * 0.25
    a0 = jnp.abs(s0r)
    m_old = m_s[0, 0]
    m_new = jnp.maximum(m_old, jnp.max(a0))
    scale = jnp.exp(m_old - m_new)
    ew = jnp.exp(a0 - m_new)
    d_s[0, 0] = d_s[0, 0] * scale + jnp.sum(ew)
    acc_ref[...] = acc_ref[...] * scale + jnp.dot(
        jnp.sign(s0r) * ew, xc, preferred_element_type=jnp.float32)
    m_s[0, 0] = m_new

    @pl.when(r == nr - 1)
    def _():
        s00 = lax.dot_general(q0, k0, (((1,), (1,)), ((), ())),
                              preferred_element_type=jnp.float32)[0, 0] * 0.25
        a00 = jnp.abs(s00)
        m_old2 = m_s[0, 0]
        m_f = jnp.maximum(m_old2, a00)
        sc2 = jnp.exp(m_old2 - m_f)
        e00 = jnp.exp(a00 - m_f)
        d_f = d_s[0, 0] * sc2 + e00
        acc_f = acc_ref[...] * sc2 + jnp.sign(s00) * e00 * a_row
        a_out = _ln(a_row + acc_f / d_f, nw, nb)
        if final:
            vout_ref[0, 0:1, :] = a_out
        else:
            aout_ref[0] = a_out


def _layer(vmain, anchor, u, v, nw, nb, final=False, prep=None):
    b, s, _ = vmain.shape
    r = s // BLK
    grid = (b, r)
    if final:
        out_specs = pl.BlockSpec((1, s + 1, DIM), lambda bi, ri: (bi, 0, 0))
        out_shape = jax.ShapeDtypeStruct((b, s + 1, DIM), jnp.float32)
    else:
        out_specs = [
            pl.BlockSpec((1, BLK, DIM), lambda bi, ri: (bi, ri, 0)),
            pl.BlockSpec((1, 1, DIM), lambda bi, ri: (bi, 0, 0)),
        ]
        out_shape = [
            jax.ShapeDtypeStruct((b, s, DIM), jnp.float32),
            jax.ShapeDtypeStruct((b, 1, DIM), jnp.float32),
        ]
    panel_specs = [
        pl.BlockSpec((1, HALF, DIM),
                     lambda bi, ri, _k=k, _nh=s // HALF:
                     (bi, jnp.clip(ri * (BLK // HALF) - 1 + _k, 0, _nh - 1), 0))
        for k in range(NPANEL)
    ]
    first = prep is not None
    extra_in = []
    extra_specs = []
    if first:
        pos, inw, inb = prep
        extra_in = [pos] * NPANEL + [inw, inb]
        extra_specs = [
            pl.BlockSpec((HALF, DIM),
                         lambda bi, ri, _k=k, _nh=s // HALF:
                         (jnp.clip(ri * (BLK // HALF) - 1 + _k, 0, _nh - 1), 0))
            for k in range(NPANEL)
        ] + [
            pl.BlockSpec((1, DIM), lambda bi, ri: (0, 0)),
            pl.BlockSpec((1, DIM), lambda bi, ri: (0, 0)),
        ]
    return pl.pallas_call(
        functools.partial(_layer_body, final=final, first=first),
        grid=grid,
        in_specs=panel_specs + extra_specs + [
            pl.BlockSpec((1, 1, DIM), lambda bi, ri: (bi, 0, 0)),
            pl.BlockSpec((DIM, RANK), lambda bi, ri: (0, 0)),
            pl.BlockSpec((DIM, RANK), lambda bi, ri: (0, 0)),
            pl.BlockSpec((1, DIM), lambda bi, ri: (0, 0)),
            pl.BlockSpec((1, DIM), lambda bi, ri: (0, 0)),
        ],
        out_specs=out_specs,
        out_shape=out_shape,
        scratch_shapes=(
            [pltpu.SMEM((1, 1), jnp.float32),
             pltpu.SMEM((1, 1), jnp.float32),
             pltpu.VMEM((1, DIM), jnp.float32)]
            + ([pltpu.VMEM((1, DIM), jnp.float32)] if final else [])
        ),
        compiler_params=pltpu.CompilerParams(
            dimension_semantics=("arbitrary", "arbitrary")),
    )(*([vmain] * NPANEL), *extra_in, anchor, u, v, nw, nb)


def kernel(input_ids, tok_emb, pos_emb, anchor_state, anchor_val, in_norm_w,
           in_norm_b, sp_w, sp_b, U, V, norm_w, norm_b):
    del anchor_state, sp_w, sp_b
    bsz, seq = input_ids.shape
    ids = input_ids.reshape(-1).astype(jnp.int32)
    emb = _sc_gather(tok_emb, ids)
    vmain = emb.reshape(bsz, seq, DIM)
    anchor = jnp.broadcast_to(anchor_val.reshape(1, 1, DIM), (bsz, 1, DIM))
    anchor = jnp.asarray(anchor)
    nlayers = U.shape[0]
    prep = (pos_emb[:seq], in_norm_w.reshape(1, DIM), in_norm_b.reshape(1, DIM))
    for l in range(nlayers - 1):
        vmain, anchor = _layer(vmain, anchor, U[l, 0], V[l, 0],
                               norm_w[l].reshape(1, DIM),
                               norm_b[l].reshape(1, DIM),
                               prep=prep if l == 0 else None)
    l = nlayers - 1
    return _layer(vmain, anchor, U[l, 0], V[l, 0],
                  norm_w[l].reshape(1, DIM), norm_b[l].reshape(1, DIM),
                  final=True, prep=prep if nlayers == 1 else None)


# speedup vs baseline: 2.7169x; 1.0741x over previous
import functools

import jax
import jax.numpy as jnp
from jax import lax
from jax.experimental import pallas as pl
from jax.experimental.pallas import tpu as pltpu
from jax.experimental.pallas import tpu_sc as plsc

DIM = 768
RANK = 16
WINDOW = 64
BLK = 512
HALF = 64
SUB = 128
WIDTH = BLK + 2 * HALF
NPANEL = WIDTH // HALF
PBLK = 512
EPS = 1e-5


def _ln(x, w, b):
    mu = jnp.mean(x, axis=-1, keepdims=True)
    var = jnp.mean((x - mu) ** 2, axis=-1, keepdims=True)
    return (x - mu) * lax.rsqrt(var + EPS) * w + b


def _sc_gather(table, ids_flat):
    info = plsc.get_sparse_core_info()
    nw = info.num_cores * info.num_subcores
    n = ids_flat.shape[0]
    per_w = n // nw
    mesh = plsc.VectorSubcoreMesh(core_axis_name="c", subcore_axis_name="s")

    @functools.partial(
        pl.kernel,
        mesh=mesh,
        out_type=jax.ShapeDtypeStruct((n, DIM), jnp.float32),
        scratch_types=[
            pltpu.VMEM((per_w,), jnp.int32),
            pltpu.VMEM((per_w, DIM), jnp.float32),
            pltpu.SemaphoreType.DMA,
        ],
    )
    def k(table_hbm, idx_hbm, out_hbm, idx_v, rows_v, sem):
        wid = lax.axis_index("s") * info.num_cores + lax.axis_index("c")
        base = wid * per_w
        pltpu.sync_copy(idx_hbm.at[pl.ds(base, per_w)], idx_v)
        pltpu.async_copy(table_hbm.at[idx_v], rows_v, sem).wait()
        pltpu.sync_copy(rows_v, out_hbm.at[pl.ds(base, per_w)])

    return k(table, ids_flat)


def _prep_body(emb_ref, pos_ref, w_ref, b_ref, out_ref):
    x = emb_ref[...] + pos_ref[...]
    out_ref[...] = _ln(x, w_ref[...], b_ref[...])


def _prep(emb, pos, w, b):
    n = emb.shape[0]
    s = pos.shape[0]
    grid = (n // PBLK,)
    return pl.pallas_call(
        _prep_body,
        grid=grid,
        in_specs=[
            pl.BlockSpec((PBLK, DIM), lambda i: (i, 0)),
            pl.BlockSpec((PBLK, DIM), lambda i: (i % (s // PBLK), 0)),
            pl.BlockSpec((1, DIM), lambda i: (0, 0)),
            pl.BlockSpec((1, DIM), lambda i: (0, 0)),
        ],
        out_specs=pl.BlockSpec((PBLK, DIM), lambda i: (i, 0)),
        out_shape=jax.ShapeDtypeStruct((n, DIM), jnp.float32),
    )(emb, pos, w, b)


def _layer_body(*refs, final, first):
    refs = list(refs)
    panel_refs = [refs.pop(0) for _ in range(NPANEL)]
    if first:
        pos_refs = [refs.pop(0) for _ in range(NPANEL)]
        inw_ref = refs.pop(0)
        inb_ref = refs.pop(0)
    if final:
        (anc_ref, u_ref, v_ref, nw_ref, nb_ref,
         vout_ref, m_s, d_s, acc_ref, prev_ref) = refs
        aout_ref = None
    else:
        (anc_ref, u_ref, v_ref, nw_ref, nb_ref,
         vout_ref, aout_ref, m_s, d_s, acc_ref) = refs
    r = pl.program_id(1)
    nr = pl.num_programs(1)
    s_tokens = nr * BLK

    if first:
        panels = [
            _ln(p[0] + q[...], inw_ref[...], inb_ref[...])
            for p, q in zip(panel_refs, pos_refs)
        ]
    else:
        panels = [p[0] for p in panel_refs]
    vcat = jnp.concatenate(panels, axis=0)
    xc = vcat[HALF:HALF + BLK]
    a_row = anc_ref[0]

    u = u_ref[...]
    v = v_ref[...]
    q = jnp.dot(xc, u, preferred_element_type=jnp.float32)
    kcat = jnp.dot(vcat, v, preferred_element_type=jnp.float32)
    k0 = jnp.dot(a_row, v, preferred_element_type=jnp.float32)

    nw = nw_ref[...]
    nb = nb_ref[...]

    sub_parts = []
    for t in range(BLK // SUB):
        off = SUB * t
        win = 2 * SUB
        q_t = q[off:off + SUB]
        k_t = kcat[off:off + win]
        v_t = vcat[off:off + win]
        scores = lax.dot_general(q_t, k_t, (((1,), (1,)), ((), ())),
                                 preferred_element_type=jnp.float32) * 0.25
        s0 = lax.dot_general(q_t, k0, (((1,), (1,)), ((), ())),
                             preferred_element_type=jnp.float32) * 0.25
        io = lax.broadcasted_iota(jnp.int32, (SUB, win), 0)
        jo = lax.broadcasted_iota(jnp.int32, (SUB, win), 1)
        jj = r * BLK - HALF + off + jo
        valid = (jnp.abs(io - jo + HALF) <= WINDOW) & (jj >= 0) & (jj < s_tokens)

        asc = jnp.abs(scores)
        m_row = jnp.maximum(
            jnp.max(jnp.where(valid, asc, -jnp.inf), axis=1, keepdims=True),
            jnp.abs(s0))
        e = jnp.where(valid, jnp.exp(asc - m_row), 0.0)
        e0 = jnp.exp(jnp.abs(s0) - m_row)
        denom = jnp.sum(e, axis=1, keepdims=True) + e0
        wgt = jnp.sign(scores) * (e / denom)
        w0 = jnp.sign(s0) * (e0 / denom)

        delta = jnp.dot(wgt, v_t, preferred_element_type=jnp.float32) + w0 * a_row
        sub_parts.append(_ln(xc[off:off + SUB] + delta, nw, nb))
    y = jnp.concatenate(sub_parts, axis=0)
    if final:
        shifted = jnp.concatenate([prev_ref[...], y[:BLK - 1]], axis=0)
        prev_ref[...] = y[BLK - 1:]
        vout_ref[0, pl.ds(r * BLK, BLK), :] = shifted
        @pl.when(r == pl.num_programs(1) - 1)
        def _():
            vout_ref[0, BLK * pl.num_programs(1):, :] = y[BLK - 1:]
    else:
        vout_ref[0] = y

    @pl.when(r == 0)
    def _():
        m_s[0, 0] = -jnp.inf
        d_s[0, 0] = 0.0
        acc_ref[...] = jnp.zeros_like(acc_ref)

    q0 = jnp.dot(a_row, u, preferred_element_type=jnp.float32)
    kc = kcat[HALF:HALF + BLK]
    s0r = lax.dot_general(q0, kc, (((1,), (1,)), ((), ())),
                          preferred_element_type=jnp.float32) * 0.25
    a0 = jnp.abs(s0r)
    m_old = m_s[0, 0]
    m_new = jnp.maximum(m_old, jnp.max(a0))
    scale = jnp.exp(m_old - m_new)
    ew = jnp.exp(a0 - m_new)
    d_s[0, 0] = d_s[0, 0] * scale + jnp.sum(ew)
    acc_ref[...] = acc_ref[...] * scale + jnp.dot(
        jnp.sign(s0r) * ew, xc, preferred_element_type=jnp.float32)
    m_s[0, 0] = m_new

    @pl.when(r == nr - 1)
    def _():
        s00 = lax.dot_general(q0, k0, (((1,), (1,)), ((), ())),
                              preferred_element_type=jnp.float32)[0, 0] * 0.25
        a00 = jnp.abs(s00)
        m_old2 = m_s[0, 0]
        m_f = jnp.maximum(m_old2, a00)
        sc2 = jnp.exp(m_old2 - m_f)
        e00 = jnp.exp(a00 - m_f)
        d_f = d_s[0, 0] * sc2 + e00
        acc_f = acc_ref[...] * sc2 + jnp.sign(s00) * e00 * a_row
        a_out = _ln(a_row + acc_f / d_f, nw, nb)
        if final:
            vout_ref[0, 0:1, :] = a_out
        else:
            aout_ref[0] = a_out


def _layer(vmain, anchor, u, v, nw, nb, final=False, prep=None):
    b, s, _ = vmain.shape
    r = s // BLK
    grid = (b, r)
    if final:
        out_specs = pl.BlockSpec((1, s + 1, DIM), lambda bi, ri: (bi, 0, 0))
        out_shape = jax.ShapeDtypeStruct((b, s + 1, DIM), jnp.float32)
    else:
        out_specs = [
            pl.BlockSpec((1, BLK, DIM), lambda bi, ri: (bi, ri, 0)),
            pl.BlockSpec((1, 1, DIM), lambda bi, ri: (bi, 0, 0)),
        ]
        out_shape = [
            jax.ShapeDtypeStruct((b, s, DIM), jnp.float32),
            jax.ShapeDtypeStruct((b, 1, DIM), jnp.float32),
        ]
    panel_specs = [
        pl.BlockSpec((1, HALF, DIM),
                     lambda bi, ri, _k=k, _nh=s // HALF:
                     (bi, jnp.clip(ri * (BLK // HALF) - 1 + _k, 0, _nh - 1), 0))
        for k in range(NPANEL)
    ]
    first = prep is not None
    extra_in = []
    extra_specs = []
    if first:
        pos, inw, inb = prep
        extra_in = [pos] * NPANEL + [inw, inb]
        extra_specs = [
            pl.BlockSpec((HALF, DIM),
                         lambda bi, ri, _k=k, _nh=s // HALF:
                         (jnp.clip(ri * (BLK // HALF) - 1 + _k, 0, _nh - 1), 0))
            for k in range(NPANEL)
        ] + [
            pl.BlockSpec((1, DIM), lambda bi, ri: (0, 0)),
            pl.BlockSpec((1, DIM), lambda bi, ri: (0, 0)),
        ]
    return pl.pallas_call(
        functools.partial(_layer_body, final=final, first=first),
        grid=grid,
        in_specs=panel_specs + extra_specs + [
            pl.BlockSpec((1, 1, DIM), lambda bi, ri: (bi, 0, 0)),
            pl.BlockSpec((DIM, RANK), lambda bi, ri: (0, 0)),
            pl.BlockSpec((DIM, RANK), lambda bi, ri: (0, 0)),
            pl.BlockSpec((1, DIM), lambda bi, ri: (0, 0)),
            pl.BlockSpec((1, DIM), lambda bi, ri: (0, 0)),
        ],
        out_specs=out_specs,
        out_shape=out_shape,
        scratch_shapes=(
            [pltpu.SMEM((1, 1), jnp.float32),
             pltpu.SMEM((1, 1), jnp.float32),
             pltpu.VMEM((1, DIM), jnp.float32)]
            + ([pltpu.VMEM((1, DIM), jnp.float32)] if final else [])
        ),
        compiler_params=pltpu.CompilerParams(
            dimension_semantics=("arbitrary", "arbitrary")),
    )(*([vmain] * NPANEL), *extra_in, anchor, u, v, nw, nb)


def kernel(input_ids, tok_emb, pos_emb, anchor_state, anchor_val, in_norm_w,
           in_norm_b, sp_w, sp_b, U, V, norm_w, norm_b):
    del anchor_state, sp_w, sp_b
    bsz, seq = input_ids.shape
    ids = input_ids.reshape(-1).astype(jnp.int32)
    emb = _sc_gather(tok_emb, ids)
    vmain = emb.reshape(bsz, seq, DIM)
    anchor = jnp.broadcast_to(anchor_val.reshape(1, 1, DIM), (bsz, 1, DIM))
    anchor = jnp.asarray(anchor)
    nlayers = U.shape[0]
    prep = (pos_emb[:seq], in_norm_w.reshape(1, DIM), in_norm_b.reshape(1, DIM))
    for l in range(nlayers - 1):
        vmain, anchor = _layer(vmain, anchor, U[l, 0], V[l, 0],
                               norm_w[l].reshape(1, DIM),
                               norm_b[l].reshape(1, DIM),
                               prep=prep if l == 0 else None)
    l = nlayers - 1
    return _layer(vmain, anchor, U[l, 0], V[l, 0],
                  norm_w[l].reshape(1, DIM), norm_b[l].reshape(1, DIM),
                  final=True, prep=prep if nlayers == 1 else None)


# speedup vs baseline: 2.8014x; 1.0311x over previous
import functools

import jax
import jax.numpy as jnp
from jax import lax
from jax.experimental import pallas as pl
from jax.experimental.pallas import tpu as pltpu
from jax.experimental.pallas import tpu_sc as plsc

DIM = 768
RANK = 16
WINDOW = 64
BLK = 1024
HALF = 64
SUB = 128
WIDTH = BLK + 2 * HALF
NPANEL = WIDTH // HALF
PBLK = 1024
EPS = 1e-5


def _ln(x, w, b):
    mu = jnp.mean(x, axis=-1, keepdims=True)
    var = jnp.mean((x - mu) ** 2, axis=-1, keepdims=True)
    return (x - mu) * lax.rsqrt(var + EPS) * w + b


def _sc_gather(table, ids_flat):
    info = plsc.get_sparse_core_info()
    nw = info.num_cores * info.num_subcores
    n = ids_flat.shape[0]
    per_w = n // nw
    mesh = plsc.VectorSubcoreMesh(core_axis_name="c", subcore_axis_name="s")

    @functools.partial(
        pl.kernel,
        mesh=mesh,
        out_type=jax.ShapeDtypeStruct((n, DIM), jnp.float32),
        scratch_types=[
            pltpu.VMEM((per_w,), jnp.int32),
            pltpu.VMEM((per_w, DIM), jnp.float32),
            pltpu.SemaphoreType.DMA,
        ],
    )
    def k(table_hbm, idx_hbm, out_hbm, idx_v, rows_v, sem):
        wid = lax.axis_index("s") * info.num_cores + lax.axis_index("c")
        base = wid * per_w
        pltpu.sync_copy(idx_hbm.at[pl.ds(base, per_w)], idx_v)
        pltpu.async_copy(table_hbm.at[idx_v], rows_v, sem).wait()
        pltpu.sync_copy(rows_v, out_hbm.at[pl.ds(base, per_w)])

    return k(table, ids_flat)


def _prep_body(emb_ref, pos_ref, w_ref, b_ref, out_ref):
    x = emb_ref[...] + pos_ref[...]
    out_ref[...] = _ln(x, w_ref[...], b_ref[...])


def _prep(emb, pos, w, b):
    n = emb.shape[0]
    s = pos.shape[0]
    grid = (n // PBLK,)
    return pl.pallas_call(
        _prep_body,
        grid=grid,
        in_specs=[
            pl.BlockSpec((PBLK, DIM), lambda i: (i, 0)),
            pl.BlockSpec((PBLK, DIM), lambda i: (i % (s // PBLK), 0)),
            pl.BlockSpec((1, DIM), lambda i: (0, 0)),
            pl.BlockSpec((1, DIM), lambda i: (0, 0)),
        ],
        out_specs=pl.BlockSpec((PBLK, DIM), lambda i: (i, 0)),
        out_shape=jax.ShapeDtypeStruct((n, DIM), jnp.float32),
    )(emb, pos, w, b)


def _layer_body(*refs, final, first):
    refs = list(refs)
    panel_refs = [refs.pop(0) for _ in range(NPANEL)]
    if first:
        pos_refs = [refs.pop(0) for _ in range(NPANEL)]
        inw_ref = refs.pop(0)
        inb_ref = refs.pop(0)
    if final:
        (anc_ref, u_ref, v_ref, nw_ref, nb_ref,
         vout_ref, m_s, d_s, acc_ref, prev_ref) = refs
        aout_ref = None
    else:
        (anc_ref, u_ref, v_ref, nw_ref, nb_ref,
         vout_ref, aout_ref, m_s, d_s, acc_ref) = refs
    r = pl.program_id(1)
    nr = pl.num_programs(1)
    s_tokens = nr * BLK

    if first:
        panels = [
            _ln(p[0] + q[...], inw_ref[...], inb_ref[...])
            for p, q in zip(panel_refs, pos_refs)
        ]
    else:
        panels = [p[0] for p in panel_refs]
    vcat = jnp.concatenate(panels, axis=0)
    xc = vcat[HALF:HALF + BLK]
    a_row = anc_ref[0]

    u = u_ref[...]
    v = v_ref[...]
    q = jnp.dot(xc, u, preferred_element_type=jnp.float32)
    kcat = jnp.dot(vcat, v, preferred_element_type=jnp.float32)
    k0 = jnp.dot(a_row, v, preferred_element_type=jnp.float32)

    nw = nw_ref[...]
    nb = nb_ref[...]

    sub_parts = []
    for t in range(BLK // SUB):
        off = SUB * t
        win = 2 * SUB
        q_t = q[off:off + SUB]
        k_t = kcat[off:off + win]
        v_t = vcat[off:off + win]
        scores = lax.dot_general(q_t, k_t, (((1,), (1,)), ((), ())),
                                 preferred_element_type=jnp.float32) * 0.25
        s0 = lax.dot_general(q_t, k0, (((1,), (1,)), ((), ())),
                             preferred_element_type=jnp.float32) * 0.25
        io = lax.broadcasted_iota(jnp.int32, (SUB, win), 0)
        jo = lax.broadcasted_iota(jnp.int32, (SUB, win), 1)
        jj = r * BLK - HALF + off + jo
        valid = (jnp.abs(io - jo + HALF) <= WINDOW) & (jj >= 0) & (jj < s_tokens)

        asc = jnp.abs(scores)
        m_row = jnp.maximum(
            jnp.max(jnp.where(valid, asc, -jnp.inf), axis=1, keepdims=True),
            jnp.abs(s0))
        e = jnp.where(valid, jnp.exp(asc - m_row), 0.0)
        e0 = jnp.exp(jnp.abs(s0) - m_row)
        denom = jnp.sum(e, axis=1, keepdims=True) + e0
        wgt = jnp.sign(scores) * (e / denom)
        w0 = jnp.sign(s0) * (e0 / denom)

        delta = jnp.dot(wgt, v_t, preferred_element_type=jnp.float32) + w0 * a_row
        sub_parts.append(_ln(xc[off:off + SUB] + delta, nw, nb))
    y = jnp.concatenate(sub_parts, axis=0)
    if final:
        shifted = jnp.concatenate([prev_ref[...], y[:BLK - 1]], axis=0)
        prev_ref[...] = y[BLK - 1:]
        vout_ref[0, pl.ds(r * BLK, BLK), :] = shifted
        @pl.when(r == pl.num_programs(1) - 1)
        def _():
            vout_ref[0, BLK * pl.num_programs(1):, :] = y[BLK - 1:]
    else:
        vout_ref[0] = y

    @pl.when(r == 0)
    def _():
        m_s[0, 0] = -jnp.inf
        d_s[0, 0] = 0.0
        acc_ref[...] = jnp.zeros_like(acc_ref)

    q0 = jnp.dot(a_row, u, preferred_element_type=jnp.float32)
    kc = kcat[HALF:HALF + BLK]
    s0r = lax.dot_general(q0, kc, (((1,), (1,)), ((), ())),
                          preferred_element_type=jnp.float32) * 0.25
    a0 = jnp.abs(s0r)
    m_old = m_s[0, 0]
    m_new = jnp.maximum(m_old, jnp.max(a0))
    scale = jnp.exp(m_old - m_new)
    ew = jnp.exp(a0 - m_new)
    d_s[0, 0] = d_s[0, 0] * scale + jnp.sum(ew)
    acc_ref[...] = acc_ref[...] * scale + jnp.dot(
        jnp.sign(s0r) * ew, xc, preferred_element_type=jnp.float32)
    m_s[0, 0] = m_new

    @pl.when(r == nr - 1)
    def _():
        s00 = lax.dot_general(q0, k0, (((1,), (1,)), ((), ())),
                              preferred_element_type=jnp.float32)[0, 0] * 0.25
        a00 = jnp.abs(s00)
        m_old2 = m_s[0, 0]
        m_f = jnp.maximum(m_old2, a00)
        sc2 = jnp.exp(m_old2 - m_f)
        e00 = jnp.exp(a00 - m_f)
        d_f = d_s[0, 0] * sc2 + e00
        acc_f = acc_ref[...] * sc2 + jnp.sign(s00) * e00 * a_row
        a_out = _ln(a_row + acc_f / d_f, nw, nb)
        if final:
            vout_ref[0, 0:1, :] = a_out
        else:
            aout_ref[0] = a_out


def _layer(vmain, anchor, u, v, nw, nb, final=False, prep=None):
    b, s, _ = vmain.shape
    r = s // BLK
    grid = (b, r)
    if final:
        out_specs = pl.BlockSpec((1, s + 1, DIM), lambda bi, ri: (bi, 0, 0))
        out_shape = jax.ShapeDtypeStruct((b, s + 1, DIM), jnp.float32)
    else:
        out_specs = [
            pl.BlockSpec((1, BLK, DIM), lambda bi, ri: (bi, ri, 0)),
            pl.BlockSpec((1, 1, DIM), lambda bi, ri: (bi, 0, 0)),
        ]
        out_shape = [
            jax.ShapeDtypeStruct((b, s, DIM), jnp.float32),
            jax.ShapeDtypeStruct((b, 1, DIM), jnp.float32),
        ]
    panel_specs = [
        pl.BlockSpec((1, HALF, DIM),
                     lambda bi, ri, _k=k, _nh=s // HALF:
                     (bi, jnp.clip(ri * (BLK // HALF) - 1 + _k, 0, _nh - 1), 0))
        for k in range(NPANEL)
    ]
    first = prep is not None
    extra_in = []
    extra_specs = []
    if first:
        pos, inw, inb = prep
        extra_in = [pos] * NPANEL + [inw, inb]
        extra_specs = [
            pl.BlockSpec((HALF, DIM),
                         lambda bi, ri, _k=k, _nh=s // HALF:
                         (jnp.clip(ri * (BLK // HALF) - 1 + _k, 0, _nh - 1), 0))
            for k in range(NPANEL)
        ] + [
            pl.BlockSpec((1, DIM), lambda bi, ri: (0, 0)),
            pl.BlockSpec((1, DIM), lambda bi, ri: (0, 0)),
        ]
    return pl.pallas_call(
        functools.partial(_layer_body, final=final, first=first),
        grid=grid,
        in_specs=panel_specs + extra_specs + [
            pl.BlockSpec((1, 1, DIM), lambda bi, ri: (bi, 0, 0)),
            pl.BlockSpec((DIM, RANK), lambda bi, ri: (0, 0)),
            pl.BlockSpec((DIM, RANK), lambda bi, ri: (0, 0)),
            pl.BlockSpec((1, DIM), lambda bi, ri: (0, 0)),
            pl.BlockSpec((1, DIM), lambda bi, ri: (0, 0)),
        ],
        out_specs=out_specs,
        out_shape=out_shape,
        scratch_shapes=(
            [pltpu.SMEM((1, 1), jnp.float32),
             pltpu.SMEM((1, 1), jnp.float32),
             pltpu.VMEM((1, DIM), jnp.float32)]
            + ([pltpu.VMEM((1, DIM), jnp.float32)] if final else [])
        ),
        compiler_params=pltpu.CompilerParams(
            dimension_semantics=("arbitrary", "arbitrary")),
    )(*([vmain] * NPANEL), *extra_in, anchor, u, v, nw, nb)


def kernel(input_ids, tok_emb, pos_emb, anchor_state, anchor_val, in_norm_w,
           in_norm_b, sp_w, sp_b, U, V, norm_w, norm_b):
    del anchor_state, sp_w, sp_b
    bsz, seq = input_ids.shape
    ids = input_ids.reshape(-1).astype(jnp.int32)
    emb = _sc_gather(tok_emb, ids)
    vmain = emb.reshape(bsz, seq, DIM)
    anchor = jnp.broadcast_to(anchor_val.reshape(1, 1, DIM), (bsz, 1, DIM))
    anchor = jnp.asarray(anchor)
    nlayers = U.shape[0]
    prep = (pos_emb[:seq], in_norm_w.reshape(1, DIM), in_norm_b.reshape(1, DIM))
    for l in range(nlayers - 1):
        vmain, anchor = _layer(vmain, anchor, U[l, 0], V[l, 0],
                               norm_w[l].reshape(1, DIM),
                               norm_b[l].reshape(1, DIM),
                               prep=prep if l == 0 else None)
    l = nlayers - 1
    return _layer(vmain, anchor, U[l, 0], V[l, 0],
                  norm_w[l].reshape(1, DIM), norm_b[l].reshape(1, DIM),
                  final=True, prep=prep if nlayers == 1 else None)


# speedup vs baseline: 2.9088x; 1.0383x over previous
import functools

import jax
import jax.numpy as jnp
from jax import lax
from jax.experimental import pallas as pl
from jax.experimental.pallas import tpu as pltpu
from jax.experimental.pallas import tpu_sc as plsc

DIM = 768
RANK = 16
WINDOW = 64
BLK = 2048
HALF = 64
SUB = 128
WIDTH = BLK + 2 * HALF
NPANEL = WIDTH // HALF
PBLK = 2048
EPS = 1e-5


def _ln(x, w, b):
    mu = jnp.mean(x, axis=-1, keepdims=True)
    var = jnp.mean((x - mu) ** 2, axis=-1, keepdims=True)
    return (x - mu) * lax.rsqrt(var + EPS) * w + b


def _sc_gather(table, ids_flat):
    info = plsc.get_sparse_core_info()
    nw = info.num_cores * info.num_subcores
    n = ids_flat.shape[0]
    per_w = n // nw
    mesh = plsc.VectorSubcoreMesh(core_axis_name="c", subcore_axis_name="s")

    @functools.partial(
        pl.kernel,
        mesh=mesh,
        out_type=jax.ShapeDtypeStruct((n, DIM), jnp.float32),
        scratch_types=[
            pltpu.VMEM((per_w,), jnp.int32),
            pltpu.VMEM((per_w, DIM), jnp.float32),
            pltpu.SemaphoreType.DMA,
        ],
    )
    def k(table_hbm, idx_hbm, out_hbm, idx_v, rows_v, sem):
        wid = lax.axis_index("s") * info.num_cores + lax.axis_index("c")
        base = wid * per_w
        pltpu.sync_copy(idx_hbm.at[pl.ds(base, per_w)], idx_v)
        pltpu.async_copy(table_hbm.at[idx_v], rows_v, sem).wait()
        pltpu.sync_copy(rows_v, out_hbm.at[pl.ds(base, per_w)])

    return k(table, ids_flat)


def _prep_body(emb_ref, pos_ref, w_ref, b_ref, out_ref):
    x = emb_ref[...] + pos_ref[...]
    out_ref[...] = _ln(x, w_ref[...], b_ref[...])


def _prep(emb, pos, w, b):
    n = emb.shape[0]
    s = pos.shape[0]
    grid = (n // PBLK,)
    return pl.pallas_call(
        _prep_body,
        grid=grid,
        in_specs=[
            pl.BlockSpec((PBLK, DIM), lambda i: (i, 0)),
            pl.BlockSpec((PBLK, DIM), lambda i: (i % (s // PBLK), 0)),
            pl.BlockSpec((1, DIM), lambda i: (0, 0)),
            pl.BlockSpec((1, DIM), lambda i: (0, 0)),
        ],
        out_specs=pl.BlockSpec((PBLK, DIM), lambda i: (i, 0)),
        out_shape=jax.ShapeDtypeStruct((n, DIM), jnp.float32),
    )(emb, pos, w, b)


def _layer_body(*refs, final, first):
    refs = list(refs)
    panel_refs = [refs.pop(0) for _ in range(NPANEL)]
    if first:
        pos_refs = [refs.pop(0) for _ in range(NPANEL)]
        inw_ref = refs.pop(0)
        inb_ref = refs.pop(0)
    if final:
        (anc_ref, u_ref, v_ref, nw_ref, nb_ref,
         vout_ref, m_s, d_s, acc_ref, prev_ref) = refs
        aout_ref = None
    else:
        (anc_ref, u_ref, v_ref, nw_ref, nb_ref,
         vout_ref, aout_ref, m_s, d_s, acc_ref) = refs
    r = pl.program_id(1)
    nr = pl.num_programs(1)
    s_tokens = nr * BLK

    if first:
        panels = [
            _ln(p[0] + q[...], inw_ref[...], inb_ref[...])
            for p, q in zip(panel_refs, pos_refs)
        ]
    else:
        panels = [p[0] for p in panel_refs]
    vcat = jnp.concatenate(panels, axis=0)
    xc = vcat[HALF:HALF + BLK]
    a_row = anc_ref[0]

    u = u_ref[...]
    v = v_ref[...]
    q = jnp.dot(xc, u, preferred_element_type=jnp.float32)
    kcat = jnp.dot(vcat, v, preferred_element_type=jnp.float32)
    k0 = jnp.dot(a_row, v, preferred_element_type=jnp.float32)

    nw = nw_ref[...]
    nb = nb_ref[...]

    sub_parts = []
    for t in range(BLK // SUB):
        off = SUB * t
        win = 2 * SUB
        q_t = q[off:off + SUB]
        k_t = kcat[off:off + win]
        v_t = vcat[off:off + win]
        scores = lax.dot_general(q_t, k_t, (((1,), (1,)), ((), ())),
                                 preferred_element_type=jnp.float32) * 0.25
        s0 = lax.dot_general(q_t, k0, (((1,), (1,)), ((), ())),
                             preferred_element_type=jnp.float32) * 0.25
        io = lax.broadcasted_iota(jnp.int32, (SUB, win), 0)
        jo = lax.broadcasted_iota(jnp.int32, (SUB, win), 1)
        jj = r * BLK - HALF + off + jo
        valid = (jnp.abs(io - jo + HALF) <= WINDOW) & (jj >= 0) & (jj < s_tokens)

        asc = jnp.abs(scores)
        m_row = jnp.maximum(
            jnp.max(jnp.where(valid, asc, -jnp.inf), axis=1, keepdims=True),
            jnp.abs(s0))
        e = jnp.where(valid, jnp.exp(asc - m_row), 0.0)
        e0 = jnp.exp(jnp.abs(s0) - m_row)
        denom = jnp.sum(e, axis=1, keepdims=True) + e0
        wgt = jnp.sign(scores) * (e / denom)
        w0 = jnp.sign(s0) * (e0 / denom)

        delta = jnp.dot(wgt, v_t, preferred_element_type=jnp.float32) + w0 * a_row
        sub_parts.append(_ln(xc[off:off + SUB] + delta, nw, nb))
    y = jnp.concatenate(sub_parts, axis=0)
    if final:
        shifted = jnp.concatenate([prev_ref[...], y[:BLK - 1]], axis=0)
        prev_ref[...] = y[BLK - 1:]
        vout_ref[0, pl.ds(r * BLK, BLK), :] = shifted
        @pl.when(r == pl.num_programs(1) - 1)
        def _():
            vout_ref[0, BLK * pl.num_programs(1):, :] = y[BLK - 1:]
    else:
        vout_ref[0] = y

    @pl.when(r == 0)
    def _():
        m_s[0, 0] = -jnp.inf
        d_s[0, 0] = 0.0
        acc_ref[...] = jnp.zeros_like(acc_ref)

    q0 = jnp.dot(a_row, u, preferred_element_type=jnp.float32)
    kc = kcat[HALF:HALF + BLK]
    s0r = lax.dot_general(q0, kc, (((1,), (1,)), ((), ())),
                          preferred_element_type=jnp.float32) * 0.25
    a0 = jnp.abs(s0r)
    m_old = m_s[0, 0]
    m_new = jnp.maximum(m_old, jnp.max(a0))
    scale = jnp.exp(m_old - m_new)
    ew = jnp.exp(a0 - m_new)
    d_s[0, 0] = d_s[0, 0] * scale + jnp.sum(ew)
    acc_ref[...] = acc_ref[...] * scale + jnp.dot(
        jnp.sign(s0r) * ew, xc, preferred_element_type=jnp.float32)
    m_s[0, 0] = m_new

    @pl.when(r == nr - 1)
    def _():
        s00 = lax.dot_general(q0, k0, (((1,), (1,)), ((), ())),
                              preferred_element_type=jnp.float32)[0, 0] * 0.25
        a00 = jnp.abs(s00)
        m_old2 = m_s[0, 0]
        m_f = jnp.maximum(m_old2, a00)
        sc2 = jnp.exp(m_old2 - m_f)
        e00 = jnp.exp(a00 - m_f)
        d_f = d_s[0, 0] * sc2 + e00
        acc_f = acc_ref[...] * sc2 + jnp.sign(s00) * e00 * a_row
        a_out = _ln(a_row + acc_f / d_f, nw, nb)
        if final:
            vout_ref[0, 0:1, :] = a_out
        else:
            aout_ref[0] = a_out


def _layer(vmain, anchor, u, v, nw, nb, final=False, prep=None):
    b, s, _ = vmain.shape
    r = s // BLK
    grid = (b, r)
    if final:
        out_specs = pl.BlockSpec((1, s + 1, DIM), lambda bi, ri: (bi, 0, 0))
        out_shape = jax.ShapeDtypeStruct((b, s + 1, DIM), jnp.float32)
    else:
        out_specs = [
            pl.BlockSpec((1, BLK, DIM), lambda bi, ri: (bi, ri, 0)),
            pl.BlockSpec((1, 1, DIM), lambda bi, ri: (bi, 0, 0)),
        ]
        out_shape = [
            jax.ShapeDtypeStruct((b, s, DIM), jnp.float32),
            jax.ShapeDtypeStruct((b, 1, DIM), jnp.float32),
        ]
    panel_specs = [
        pl.BlockSpec((1, HALF, DIM),
                     lambda bi, ri, _k=k, _nh=s // HALF:
                     (bi, jnp.clip(ri * (BLK // HALF) - 1 + _k, 0, _nh - 1), 0))
        for k in range(NPANEL)
    ]
    first = prep is not None
    extra_in = []
    extra_specs = []
    if first:
        pos, inw, inb = prep
        extra_in = [pos] * NPANEL + [inw, inb]
        extra_specs = [
            pl.BlockSpec((HALF, DIM),
                         lambda bi, ri, _k=k, _nh=s // HALF:
                         (jnp.clip(ri * (BLK // HALF) - 1 + _k, 0, _nh - 1), 0))
            for k in range(NPANEL)
        ] + [
            pl.BlockSpec((1, DIM), lambda bi, ri: (0, 0)),
            pl.BlockSpec((1, DIM), lambda bi, ri: (0, 0)),
        ]
    return pl.pallas_call(
        functools.partial(_layer_body, final=final, first=first),
        grid=grid,
        in_specs=panel_specs + extra_specs + [
            pl.BlockSpec((1, 1, DIM), lambda bi, ri: (bi, 0, 0)),
            pl.BlockSpec((DIM, RANK), lambda bi, ri: (0, 0)),
            pl.BlockSpec((DIM, RANK), lambda bi, ri: (0, 0)),
            pl.BlockSpec((1, DIM), lambda bi, ri: (0, 0)),
            pl.BlockSpec((1, DIM), lambda bi, ri: (0, 0)),
        ],
        out_specs=out_specs,
        out_shape=out_shape,
        scratch_shapes=(
            [pltpu.SMEM((1, 1), jnp.float32),
             pltpu.SMEM((1, 1), jnp.float32),
             pltpu.VMEM((1, DIM), jnp.float32)]
            + ([pltpu.VMEM((1, DIM), jnp.float32)] if final else [])
        ),
        compiler_params=pltpu.CompilerParams(
            dimension_semantics=("arbitrary", "arbitrary")),
    )(*([vmain] * NPANEL), *extra_in, anchor, u, v, nw, nb)


def kernel(input_ids, tok_emb, pos_emb, anchor_state, anchor_val, in_norm_w,
           in_norm_b, sp_w, sp_b, U, V, norm_w, norm_b):
    del anchor_state, sp_w, sp_b
    bsz, seq = input_ids.shape
    ids = input_ids.reshape(-1).astype(jnp.int32)
    emb = _sc_gather(tok_emb, ids)
    vmain = emb.reshape(bsz, seq, DIM)
    anchor = jnp.broadcast_to(anchor_val.reshape(1, 1, DIM), (bsz, 1, DIM))
    anchor = jnp.asarray(anchor)
    nlayers = U.shape[0]
    prep = (pos_emb[:seq], in_norm_w.reshape(1, DIM), in_norm_b.reshape(1, DIM))
    for l in range(nlayers - 1):
        vmain, anchor = _layer(vmain, anchor, U[l, 0], V[l, 0],
                               norm_w[l].reshape(1, DIM),
                               norm_b[l].reshape(1, DIM),
                               prep=prep if l == 0 else None)
    l = nlayers - 1
    return _layer(vmain, anchor, U[l, 0], V[l, 0],
                  norm_w[l].reshape(1, DIM), norm_b[l].reshape(1, DIM),
                  final=True, prep=prep if nlayers == 1 else None)


# speedup vs baseline: 3.1409x; 1.0798x over previous
import functools

import jax
import jax.numpy as jnp
from jax import lax
from jax.experimental import pallas as pl
from jax.experimental.pallas import tpu as pltpu
from jax.experimental.pallas import tpu_sc as plsc

DIM = 768
RANK = 16
WINDOW = 64
HALF = 64
SUB = 128
WIN = 2 * SUB
EPS = 1e-5


def _ln(x, w, b):
    mu = jnp.mean(x, axis=-1, keepdims=True)
    var = jnp.mean((x - mu) ** 2, axis=-1, keepdims=True)
    return (x - mu) * lax.rsqrt(var + EPS) * w + b


def _sc_gather(table, ids_flat):
    info = plsc.get_sparse_core_info()
    nw = info.num_cores * info.num_subcores
    n = ids_flat.shape[0]
    per_w = n // nw
    mesh = plsc.VectorSubcoreMesh(core_axis_name="c", subcore_axis_name="s")

    @functools.partial(
        pl.kernel,
        mesh=mesh,
        out_type=jax.ShapeDtypeStruct((n, DIM), jnp.float32),
        scratch_types=[
            pltpu.VMEM((per_w,), jnp.int32),
            pltpu.VMEM((per_w, DIM), jnp.float32),
            pltpu.SemaphoreType.DMA,
        ],
    )
    def k(table_hbm, idx_hbm, out_hbm, idx_v, rows_v, sem):
        wid = lax.axis_index("s") * info.num_cores + lax.axis_index("c")
        base = wid * per_w
        pltpu.sync_copy(idx_hbm.at[pl.ds(base, per_w)], idx_v)
        pltpu.async_copy(table_hbm.at[idx_v], rows_v, sem).wait()
        pltpu.sync_copy(rows_v, out_hbm.at[pl.ds(base, per_w)])

    return k(table, ids_flat)


def _fused_body(emb_ref, pos_ref, anc_ref, u_ref, v_ref, nw_ref, nb_ref,
                inw_ref, inb_ref, out_ref):
    s = pos_ref.shape[0]
    nlayers = u_ref.shape[0]

    x = _ln(emb_ref[0] + pos_ref[...], inw_ref[...], inb_ref[...])
    a_row = anc_ref[...]

    for l in range(nlayers):
        u = u_ref[l]
        v = v_ref[l]
        nw = nw_ref[l:l + 1]
        nb = nb_ref[l:l + 1]

        q = jnp.dot(x, u, preferred_element_type=jnp.float32)
        kk = jnp.dot(x, v, preferred_element_type=jnp.float32)
        q0 = jnp.dot(a_row, u, preferred_element_type=jnp.float32)
        k0 = jnp.dot(a_row, v, preferred_element_type=jnp.float32)

        parts = []
        for t in range(s // SUB):
            off = SUB * t
            ws = min(max(off - HALF, 0), s - WIN)
            q_t = q[off:off + SUB]
            k_t = kk[ws:ws + WIN]
            v_t = x[ws:ws + WIN]
            scores = lax.dot_general(q_t, k_t, (((1,), (1,)), ((), ())),
                                     preferred_element_type=jnp.float32) * 0.25
            s0 = lax.dot_general(q_t, k0, (((1,), (1,)), ((), ())),
                                 preferred_element_type=jnp.float32) * 0.25
            io = lax.broadcasted_iota(jnp.int32, (SUB, WIN), 0)
            jo = lax.broadcasted_iota(jnp.int32, (SUB, WIN), 1)
            valid = jnp.abs(io - jo + (off - ws)) <= WINDOW

            asc = jnp.abs(scores)
            m_row = jnp.maximum(
                jnp.max(jnp.where(valid, asc, -jnp.inf), axis=1, keepdims=True),
                jnp.abs(s0))
            e = jnp.where(valid, jnp.exp(asc - m_row), 0.0)
            e0 = jnp.exp(jnp.abs(s0) - m_row)
            denom = jnp.sum(e, axis=1, keepdims=True) + e0
            wgt = jnp.sign(scores) * (e / denom)
            w0 = jnp.sign(s0) * (e0 / denom)

            delta = jnp.dot(wgt, v_t,
                            preferred_element_type=jnp.float32) + w0 * a_row
            parts.append(_ln(x[off:off + SUB] + delta, nw, nb))

        s_all = lax.dot_general(q0, kk, (((1,), (1,)), ((), ())),
                                preferred_element_type=jnp.float32) * 0.25
        s00 = lax.dot_general(q0, k0, (((1,), (1,)), ((), ())),
                              preferred_element_type=jnp.float32) * 0.25
        m0 = jnp.maximum(jnp.max(jnp.abs(s_all)), jnp.abs(s00)[0, 0])
        e_all = jnp.exp(jnp.abs(s_all) - m0)
        e00 = jnp.exp(jnp.abs(s00) - m0)
        den0 = jnp.sum(e_all) + e00[0, 0]
        delta0 = (jnp.dot(jnp.sign(s_all) * e_all, x,
                          preferred_element_type=jnp.float32)
                  + jnp.sign(s00) * e00 * a_row) / den0
        a_row = _ln(a_row + delta0, nw, nb)

        x = jnp.concatenate(parts, axis=0)

    shifted = jnp.concatenate([a_row, x[:s - 1]], axis=0)
    out_ref[0, 0:s, :] = shifted
    out_ref[0, s:s + 1, :] = x[s - 1:]


def _fused(emb, pos, anchor_row, u, v, nw, nb, inw, inb):
    b, s, _ = emb.shape
    return pl.pallas_call(
        _fused_body,
        grid=(b,),
        in_specs=[
            pl.BlockSpec((1, s, DIM), lambda bi: (bi, 0, 0)),
            pl.BlockSpec((s, DIM), lambda bi: (0, 0)),
            pl.BlockSpec((1, DIM), lambda bi: (0, 0)),
            pl.BlockSpec(u.shape, lambda bi: (0, 0, 0)),
            pl.BlockSpec(v.shape, lambda bi: (0, 0, 0)),
            pl.BlockSpec(nw.shape, lambda bi: (0, 0)),
            pl.BlockSpec(nb.shape, lambda bi: (0, 0)),
            pl.BlockSpec((1, DIM), lambda bi: (0, 0)),
            pl.BlockSpec((1, DIM), lambda bi: (0, 0)),
        ],
        out_specs=pl.BlockSpec((1, s + 1, DIM), lambda bi: (bi, 0, 0)),
        out_shape=jax.ShapeDtypeStruct((b, s + 1, DIM), jnp.float32),
        compiler_params=pltpu.CompilerParams(
            dimension_semantics=("arbitrary",)),
    )(emb, pos, anchor_row, u, v, nw, nb, inw, inb)


def kernel(input_ids, tok_emb, pos_emb, anchor_state, anchor_val, in_norm_w,
           in_norm_b, sp_w, sp_b, U, V, norm_w, norm_b):
    del anchor_state, sp_w, sp_b
    bsz, seq = input_ids.shape
    ids = input_ids.reshape(-1).astype(jnp.int32)
    emb = _sc_gather(tok_emb, ids).reshape(bsz, seq, DIM)
    return _fused(emb, pos_emb[:seq], anchor_val.reshape(1, DIM),
                  U[:, 0], V[:, 0], norm_w, norm_b,
                  in_norm_w.reshape(1, DIM), in_norm_b.reshape(1, DIM))
